# baseline (speedup 1.0000x reference)
import functools

import jax
import jax.numpy as jnp
from jax import lax
from jax.experimental import pallas as pl
from jax.experimental.pallas import tpu as pltpu
from jax.experimental.pallas import tpu_sc as plsc

_N = 10000
_E = 320000
_F = 128
_G = 20
_GP = 32
_NMP = 3
_NC = 2
_NS = 16
_L = 16
_NW = _NC * _NS
_NP = 10240
_EP = 327680
_CH = 128
_CHM = 80
_EPW = _EP // _NW
_NCHUNK = _EPW // _CH
_NCHUNKM = _EPW // _CHM
_RPS = _NP // _NS
_FW = 16
_LN2 = 0.6931471805599453
_BN = 512
_BE = 2048


def _ssp(x):
    return jnp.maximum(x, 0.0) + jnp.log1p(jnp.exp(-jnp.abs(x))) - _LN2


def _init_nodes_body(z_ref, emb_ref, h_ref):
    z = z_ref[...]
    oh = (z == lax.broadcasted_iota(jnp.int32, (_BN, 128), 1)).astype(jnp.float32)
    h_ref[...] = jnp.dot(oh, emb_ref[...], preferred_element_type=jnp.float32)


def _init_edges_body(d_ref, off_ref, w_ref, e_ref):
    d = d_ref[...]
    off = off_ref[...]
    w = w_ref[...]
    e = jnp.exp(-((d - off) ** 2) / (2.0 * w * w))
    mask = lax.broadcasted_iota(jnp.int32, e.shape, 1) < _G
    e_ref[...] = jnp.where(mask, e, 0.0)


def _node_mm_body(h_ref, w1_ref, sw1_ref):
    sw1_ref[...] = _ssp(
        jnp.dot(h_ref[...], w1_ref[...], preferred_element_type=jnp.float32))


def _edge_mm_body(e_ref, w2_ref, w4_ref, c_ref, t_ref, ew4_ref):
    e = e_ref[...]
    t_ref[...] = (_ssp(
        jnp.dot(e, w2_ref[...], preferred_element_type=jnp.float32))
        * c_ref[...]).astype(jnp.bfloat16)
    ew4_ref[...] = jnp.dot(e, w4_ref[...], preferred_element_type=jnp.float32)


def _node_upd_body(h_ref, a0_ref, a1_ref, w3_ref, hn_ref, hw3_ref):
    h = h_ref[...] + _ssp(a0_ref[...] + a1_ref[...])
    hn_ref[...] = h
    hw3_ref[...] = jnp.dot(h, w3_ref[...], preferred_element_type=jnp.float32)


def _edge_upd_body(e_ref, g_ref, ew4_ref, en_ref):
    en_ref[...] = e_ref[...] + _ssp(g_ref[...] + ew4_ref[...])


def _force_body(e_ref, wo1_ref, bo1_ref, wo2_ref, bo2_ref, uv_ref, fe_ref):
    z1 = _ssp(
        jnp.dot(e_ref[...], wo1_ref[...], preferred_element_type=jnp.float32)
        + bo1_ref[...])
    f = jnp.sum(z1 * wo2_ref[...], axis=1, keepdims=True) + bo2_ref[...]
    fe_ref[...] = f * uv_ref[...]


def _combine_body(p0_ref, p1_ref, o_ref):
    o_ref[...] = p0_ref[...] + p1_ref[...]


_MESH = plsc.VectorSubcoreMesh(
    core_axis_name="c", subcore_axis_name="s", num_cores=_NC, num_subcores=_NS)


def _zero_acc(u, acc, s, ch):
    def _zrow(r, carry):
        for j in range(_F // _L):
            u[r, pl.ds(j * _L, _L)] = jnp.zeros((_L,), jnp.float32)
        return carry
    lax.fori_loop(0, ch, _zrow, 0)

    def _zacc(i, carry):
        pltpu.sync_copy(u, acc.at[pl.ds(pl.multiple_of(s * _RPS + i * ch, 8), ch)])
        return carry
    lax.fori_loop(0, _RPS // ch, _zacc, 0)


def _acc_out(acc, out_hbm, c, s):
    def _wout(i, carry):
        r0 = s * _RPS + i * _CH
        pltpu.sync_copy(acc.at[pl.ds(pl.multiple_of(r0, 8), _CH)],
                        out_hbm.at[pl.ds(pl.multiple_of(c * _NP + r0, 8), _CH)])
        return carry
    lax.fori_loop(0, _RPS // _CH, _wout, 0)


@functools.partial(
    pl.kernel,
    out_type=jax.ShapeDtypeStruct((_NC * _NP, _F), jnp.float32),
    mesh=_MESH,
    scratch_types=[
        pltpu.VMEM((_CHM,), jnp.int32),
        pltpu.VMEM((_CHM,), jnp.int32),
        pltpu.VMEM((_CHM,), jnp.int32),
        pltpu.VMEM((_CHM,), jnp.int32),
        pltpu.VMEM((_CHM, _F), jnp.float32),
        pltpu.VMEM((_CHM, _F), jnp.float32),
        pltpu.VMEM((_CHM * (_F // 2),), jnp.int32),
        pltpu.VMEM((_CHM * (_F // 2),), jnp.int32),
        pltpu.VMEM_SHARED((_NP, _F), jnp.float32),
    ] + [pltpu.SemaphoreType.DMA] * 10,
)
def _msg_agg(sw1_hbm, t_hbm, src_hbm, dst_hbm, out_hbm,
             si0, si1, di0, di1, u0, u1, t0, t1, acc,
             smi0, smi1, smd0, smd1, smt0, smt1, smg0, smg1, sms0, sms1):
    c = lax.axis_index("c")
    s = lax.axis_index("s")
    wid = s * _NC + c
    ebase = wid * _EPW
    si = (si0, si1)
    di = (di0, di1)
    u = (u0, u1)
    t = (t0, t1)
    smi = (smi0, smi1)
    smd = (smd0, smd1)
    smt = (smt0, smt1)
    smg = (smg0, smg1)
    sms = (sms0, sms1)

    _zero_acc(u0, acc, s, _CHM)
    plsc.subcore_barrier()

    def drain_scatter(x):
        pltpu.make_async_copy(u[x], acc.at[di[x]], sms[x]).wait()

    def issue(x, g, drain):
        if drain:
            drain_scatter(x)
        base = ebase + g * _CHM
        pltpu.async_copy(src_hbm.at[pl.ds(base, _CHM)], si[x], smi[x])
        pltpu.async_copy(dst_hbm.at[pl.ds(base, _CHM)], di[x], smd[x])
        pltpu.async_copy(
            t_hbm.at[pl.ds(pl.multiple_of(base * (_F // 2), 8), _CHM * (_F // 2))],
            t[x], smt[x])

    def gather(x):
        pltpu.make_async_copy(src_hbm.at[pl.ds(0, _CHM)], si[x], smi[x]).wait()
        pltpu.async_copy(sw1_hbm.at[si[x]], u[x], smg[x])

    def process(x):
        pltpu.make_async_copy(sw1_hbm.at[si[x]], u[x], smg[x]).wait()
        pltpu.make_async_copy(
            t_hbm.at[pl.ds(0, _CHM * (_F // 2))], t[x], smt[x]).wait()

        def _mrow(r, cc):
            for j in range(_F // (2 * _L)):
                toff = pl.multiple_of(r * (_F // 2) + j * _L, 8)
                w = t[x][pl.ds(toff, _L)]
                ta = lax.bitcast_convert_type(w << 16, jnp.float32)
                tb = lax.bitcast_convert_type(w & jnp.int32(-65536), jnp.float32)
                sa = pl.ds(j * (2 * _L), _L)
                sb = pl.ds(j * (2 * _L) + _L, _L)
                u[x][r, sa] = u[x][r, sa] * ta
                u[x][r, sb] = u[x][r, sb] * tb
            return cc
        lax.fori_loop(0, _CHM, _mrow, 0)
        pltpu.make_async_copy(dst_hbm.at[pl.ds(0, _CHM)], di[x], smd[x]).wait()
        pltpu.async_copy(u[x], acc.at[di[x]], sms[x], add=True)

    issue(0, 0, False)
    gather(0)
    issue(1, 1, False)

    def body(i, carry):
        g = 2 * i
        process(0)
        gather(1)
        issue(0, g + 2, True)
        process(1)
        gather(0)
        issue(1, g + 3, True)
        return carry
    lax.fori_loop(0, _NCHUNKM // 2 - 1, body, 0)
    process(0)
    gather(1)
    process(1)
    drain_scatter(0)
    drain_scatter(1)

    plsc.subcore_barrier()
    _acc_out(acc, out_hbm, c, s)


@functools.partial(
    pl.kernel,
    out_type=jax.ShapeDtypeStruct((_EP, _GP), jnp.float32),
    mesh=_MESH,
    scratch_types=[
        pltpu.VMEM((_CH,), jnp.int32),
        pltpu.VMEM((_CH,), jnp.int32),
        pltpu.VMEM((_CH,), jnp.int32),
        pltpu.VMEM((_CH,), jnp.int32),
        pltpu.VMEM((_CH, _F), jnp.float32),
        pltpu.VMEM((_CH, _F), jnp.float32),
        pltpu.VMEM((_CH, _F), jnp.float32),
        pltpu.VMEM((_CH, _F), jnp.float32),
        pltpu.VMEM((_CH, _GP), jnp.float32),
        pltpu.VMEM((_CH, _GP), jnp.float32),
    ] + [pltpu.SemaphoreType.DMA] * 8,
)
def _gather2(hw3_hbm, src_hbm, dst_hbm, out_hbm,
             si0, si1, di0, di1, a0, a1, b0, b1, gn0, gn1,
             smi0, smi1, smd0, smd1, smga0, smga1, smgb0, smgb1):
    c = lax.axis_index("c")
    s = lax.axis_index("s")
    wid = s * _NC + c
    ebase = wid * _EPW
    si = (si0, si1)
    di = (di0, di1)
    a = (a0, a1)
    b = (b0, b1)
    gn = (gn0, gn1)
    smi = (smi0, smi1)
    smd = (smd0, smd1)
    smga = (smga0, smga1)
    smgb = (smgb0, smgb1)

    def issue(x, g):
        base = ebase + g * _CH
        pltpu.async_copy(src_hbm.at[pl.ds(base, _CH)], si[x], smi[x])
        pltpu.async_copy(dst_hbm.at[pl.ds(base, _CH)], di[x], smd[x])

    def gather(x):
        pltpu.make_async_copy(src_hbm.at[pl.ds(0, _CH)], si[x], smi[x]).wait()
        pltpu.async_copy(hw3_hbm.at[si[x]], a[x], smga[x])
        pltpu.make_async_copy(dst_hbm.at[pl.ds(0, _CH)], di[x], smd[x]).wait()
        pltpu.async_copy(hw3_hbm.at[di[x]], b[x], smgb[x])

    def process(x, g):
        base = ebase + g * _CH
        pltpu.make_async_copy(hw3_hbm.at[si[x]], a[x], smga[x]).wait()
        pltpu.make_async_copy(hw3_hbm.at[di[x]], b[x], smgb[x]).wait()

        def _arow(r2, cc):
            for rr in range(2):
                r = r2 * 2 + rr
                for j in range(_GP // _L):
                    sl = pl.ds(j * _L, _L)
                    gn[x][r, sl] = a[x][r, sl] + b[x][r, sl]
            return cc
        lax.fori_loop(0, _CH // 2, _arow, 0)
        pltpu.sync_copy(gn[x], out_hbm.at[pl.ds(pl.multiple_of(base, 8), _CH)])

    issue(0, 0)
    gather(0)
    issue(1, 1)

    def body(i, carry):
        g = 2 * i
        process(0, g)
        gather(1)
        issue(0, g + 2)
        process(1, g + 1)
        gather(0)
        issue(1, g + 3)
        return carry
    lax.fori_loop(0, _NCHUNK // 2 - 1, body, 0)
    process(0, _NCHUNK - 2)
    gather(1)
    process(1, _NCHUNK - 1)


@functools.partial(
    pl.kernel,
    out_type=jax.ShapeDtypeStruct((_NC * _NP, _F), jnp.float32),
    mesh=_MESH,
    scratch_types=[
        pltpu.VMEM((_CH,), jnp.int32),
        pltpu.VMEM((_CH,), jnp.int32),
        pltpu.VMEM((_CH // 8, _F), jnp.float32),
        pltpu.VMEM((_CH // 8, _F), jnp.float32),
        pltpu.VMEM((_CH, _F), jnp.float32),
        pltpu.VMEM((_CH, _F), jnp.float32),
        pltpu.VMEM_SHARED((_NP, _F), jnp.float32),
    ] + [pltpu.SemaphoreType.DMA] * 6,
)
def _scatter_f(fe8_hbm, dst_hbm, out_hbm, di0, di1, fb0, fb1, u0, u1, acc,
               smd0, smd1, smf0, smf1, sms0, sms1):
    c = lax.axis_index("c")
    s = lax.axis_index("s")
    wid = s * _NC + c
    di = (di0, di1)
    fb = (fb0, fb1)
    u = (u0, u1)
    smd = (smd0, smd1)
    smf = (smf0, smf1)
    sms = (sms0, sms1)

    _zero_acc(u0, acc, s, _CH)

    def _zrow(r, carry):
        for j in range(_F // _L):
            u1[r, pl.ds(j * _L, _L)] = jnp.zeros((_L,), jnp.float32)
        return carry
    lax.fori_loop(0, _CH, _zrow, 0)
    plsc.subcore_barrier()

    ebase = wid * _EPW

    def drain_scatter(x):
        pltpu.make_async_copy(u[x], acc.at[di[x]], sms[x]).wait()

    def issue(x, g, drain):
        if drain:
            drain_scatter(x)
        base = ebase + g * _CH
        pltpu.async_copy(dst_hbm.at[pl.ds(base, _CH)], di[x], smd[x])
        pltpu.async_copy(
            fe8_hbm.at[pl.ds(pl.multiple_of(base // 8, 8), _CH // 8)], fb[x], smf[x])

    def process(x):
        pltpu.make_async_copy(fe8_hbm.at[pl.ds(0, _CH // 8)], fb[x], smf[x]).wait()

        def _expand(q, cc):
            for rr in range(8):
                u[x][q * 8 + rr, pl.ds(0, _FW)] = fb[x][q, pl.ds(rr * _FW, _FW)]
            return cc
        lax.fori_loop(0, _CH // 8, _expand, 0)
        pltpu.make_async_copy(dst_hbm.at[pl.ds(0, _CH)], di[x], smd[x]).wait()
        pltpu.async_copy(u[x], acc.at[di[x]], sms[x], add=True)

    issue(0, 0, False)
    issue(1, 1, False)

    def body(i, carry):
        g = 2 * i
        process(0)
        issue(0, g + 2, True)
        process(1)
        issue(1, g + 3, True)
        return carry
    lax.fori_loop(0, _NCHUNK // 2 - 1, body, 0)
    process(0)
    process(1)
    drain_scatter(0)
    drain_scatter(1)

    plsc.subcore_barrier()
    _acc_out(acc, out_hbm, c, s)


_NGRID = _NP // _BN
_EGRID = _EP // _BE


def _full(shape):
    return pl.BlockSpec(shape, lambda i: tuple(0 for _ in shape))


def _rows(shape):
    return pl.BlockSpec(shape, lambda i: (i,) + tuple(0 for _ in shape[1:]))


def _init_nodes(zp, embp):
    return pl.pallas_call(
        _init_nodes_body,
        grid=(_NGRID,),
        in_specs=[_rows((_BN, 1)), _full((128, _F))],
        out_specs=_rows((_BN, _F)),
        out_shape=jax.ShapeDtypeStruct((_NP, _F), jnp.float32),
    )(zp, embp)


def _init_edges(dp, offp, widp):
    return pl.pallas_call(
        _init_edges_body,
        grid=(_EGRID,),
        in_specs=[_rows((_BE, 1)), _full((1, _GP)), _full((1, _GP))],
        out_specs=_rows((_BE, _GP)),
        out_shape=jax.ShapeDtypeStruct((_EP, _GP), jnp.float32),
    )(dp, offp, widp)


def _node_mm(h, w1):
    return pl.pallas_call(
        _node_mm_body,
        grid=(_NGRID,),
        in_specs=[_rows((_BN, _F)), _full((_F, _F))],
        out_specs=_rows((_BN, _F)),
        out_shape=jax.ShapeDtypeStruct((_NP, _F), jnp.float32),
    )(h, w1)


def _edge_mm(e, w2, w4, condp):
    return pl.pallas_call(
        _edge_mm_body,
        grid=(_EGRID,),
        in_specs=[_rows((_BE, _GP)), _full((_GP, _F)), _full((_GP, _GP)),
                  _rows((_BE, 1))],
        out_specs=[_rows((_BE, _F)), _rows((_BE, _GP))],
        out_shape=[jax.ShapeDtypeStruct((_EP, _F), jnp.bfloat16),
                   jax.ShapeDtypeStruct((_EP, _GP), jnp.float32)],
    )(e, w2, w4, condp)


def _node_upd(h, aggp, w3):
    a0 = pl.BlockSpec((_BN, _F), lambda i: (i, 0))
    a1 = pl.BlockSpec((_BN, _F), lambda i: (i + _NGRID, 0))
    return pl.pallas_call(
        _node_upd_body,
        grid=(_NGRID,),
        in_specs=[_rows((_BN, _F)), a0, a1, _full((_F, _F))],
        out_specs=[_rows((_BN, _F)), _rows((_BN, _F))],
        out_shape=[jax.ShapeDtypeStruct((_NP, _F), jnp.float32),
                   jax.ShapeDtypeStruct((_NP, _F), jnp.float32)],
    )(h, aggp, aggp, w3)


def _edge_upd(e, g, ew4):
    return pl.pallas_call(
        _edge_upd_body,
        grid=(_EGRID,),
        in_specs=[_rows((_BE, _GP))] * 3,
        out_specs=_rows((_BE, _GP)),
        out_shape=jax.ShapeDtypeStruct((_EP, _GP), jnp.float32),
    )(e, g, ew4)


def _force(e, wo1, bo1, wo2t, bo2, uvp):
    return pl.pallas_call(
        _force_body,
        grid=(_EGRID,),
        in_specs=[_rows((_BE, _GP)), _full((_GP, _GP)), _full((1, _GP)),
                  _full((1, _GP)), _full((1, 1)), _rows((_BE, _FW))],
        out_specs=_rows((_BE, _FW)),
        out_shape=jax.ShapeDtypeStruct((_EP, _FW), jnp.float32),
    )(e, wo1, bo1, wo2t, bo2, uvp)


def _combine(fp):
    p0 = pl.BlockSpec((_BN, _F), lambda i: (i, 0))
    p1 = pl.BlockSpec((_BN, _F), lambda i: (i + _NGRID, 0))
    return pl.pallas_call(
        _combine_body,
        grid=(_NGRID,),
        in_specs=[p0, p1],
        out_specs=_rows((_BN, _F)),
        out_shape=jax.ShapeDtypeStruct((_NP, _F), jnp.float32),
    )(fp, fp)


def kernel(Z, edge_index, distances, unit_vecs, conductance, emb_table,
           g_offsets, g_widths, W1, W2, W3, W4, Wo1, bo1, Wo2, bo2):
    f32 = jnp.float32
    epad = _EP - _E
    npad = _NP - _N

    src = jnp.pad(edge_index[0].astype(jnp.int32), (0, epad))
    dst = jnp.pad(edge_index[1].astype(jnp.int32), (0, epad))
    dp = jnp.pad(distances.astype(f32), (0, epad),
                 constant_values=1.0).reshape(_EP, 1)
    condp = jnp.pad(conductance.astype(f32), (0, epad)).reshape(_EP, 1)
    uvp = jnp.pad(unit_vecs.astype(f32), ((0, epad), (0, _FW - 3)))
    zp = jnp.pad(Z.astype(jnp.int32), (0, npad)).reshape(_NP, 1)
    embp = jnp.pad(emb_table.astype(f32), ((0, 128 - emb_table.shape[0]), (0, 0)))
    offp = jnp.pad(g_offsets.astype(f32), (0, _GP - _G)).reshape(1, _GP)
    widp = jnp.pad(g_widths.astype(f32), (0, _GP - _G),
                   constant_values=1.0).reshape(1, _GP)
    W1 = W1.astype(f32)
    W2p = jnp.pad(W2.astype(f32), ((0, 0), (0, _GP - _G), (0, 0)))
    W3p = jnp.pad(W3.astype(f32), ((0, 0), (0, 0), (0, _F - _G)))
    W4p = jnp.pad(W4.astype(f32), ((0, 0), (0, _GP - _G), (0, _GP - _G)))
    Wo1p = jnp.pad(Wo1.astype(f32), ((0, _GP - _G), (0, _GP - _G)))
    bo1p = jnp.pad(bo1.astype(f32), (0, _GP - _G)).reshape(1, _GP)
    Wo2t = jnp.pad(Wo2.astype(f32), ((0, _GP - _G), (0, 0))).reshape(1, _GP)
    bo2p = bo2.astype(f32).reshape(1, 1)

    h = _init_nodes(zp, embp)
    e = _init_edges(dp, offp, widp)

    perm = []
    for j in range(_F // 32):
        for m in range(16):
            perm.extend([32 * j + m, 32 * j + 16 + m])
    perm = jnp.asarray(perm, jnp.int32)

    for l in range(_NMP):
        sw1 = _node_mm(h, W1[l])
        t, ew4 = _edge_mm(e, W2p[l][:, perm], W4p[l], condp)
        tp = lax.bitcast_convert_type(
            t.reshape(_EP, _F // 2, 2), jnp.int32).reshape(-1)
        aggp = _msg_agg(sw1, tp, src, dst)
        h, hw3 = _node_upd(h, aggp, W3p[l])
        g = _gather2(hw3, src, dst)
        e = _edge_upd(e, g, ew4)

    fe = _force(e, Wo1p, bo1p, Wo2t, bo2p, uvp)
    fp = _scatter_f(fe.reshape(_EP // 8, _F), dst)
    forces = _combine(fp)
    return forces[:_N, :3]


# speedup vs baseline: 1.5657x; 1.5657x over previous
import functools

import jax
import jax.numpy as jnp
from jax import lax
from jax.experimental import pallas as pl
from jax.experimental.pallas import tpu as pltpu
from jax.experimental.pallas import tpu_sc as plsc

_N = 10000
_E = 320000
_F = 128
_G = 20
_GP = 32
_NMP = 3
_NC = 2
_NS = 16
_L = 16
_NW = _NC * _NS
_NP = 10240
_EP = 327680
_CH = 128
_CHM = 64
_EPW = _EP // _NW
_NCHUNK = _EPW // _CH
_NCHUNKM = _EPW // _CHM
_RPS = _NP // _NS
_FW = 16
_LN2 = 0.6931471805599453
_BN = 512
_BE = 2048


def _ssp(x):
    return jnp.maximum(x, 0.0) + jnp.log1p(jnp.exp(-jnp.abs(x))) - _LN2


def _init_nodes_body(z_ref, emb_ref, h_ref):
    z = z_ref[...]
    oh = (z == lax.broadcasted_iota(jnp.int32, (_BN, 128), 1)).astype(jnp.float32)
    h_ref[...] = jnp.dot(oh, emb_ref[...], preferred_element_type=jnp.float32)


def _init_edges_body(d_ref, off_ref, w_ref, e_ref):
    d = d_ref[...]
    off = off_ref[...]
    w = w_ref[...]
    e = jnp.exp(-((d - off) ** 2) / (2.0 * w * w))
    mask = lax.broadcasted_iota(jnp.int32, e.shape, 1) < _G
    e_ref[...] = jnp.where(mask, e, 0.0)


def _node_mm_body(h_ref, w1_ref, sw1_ref):
    sw1_ref[...] = _ssp(
        jnp.dot(h_ref[...], w1_ref[...], preferred_element_type=jnp.float32))


def _edge_mm_body(e_ref, w2_ref, w4_ref, c_ref, t_ref, ew4_ref):
    e = e_ref[...]
    t_ref[...] = _ssp(
        jnp.dot(e, w2_ref[...], preferred_element_type=jnp.float32)) * c_ref[...]
    ew4_ref[...] = jnp.dot(e, w4_ref[...], preferred_element_type=jnp.float32)


def _node_upd_body(h_ref, a0_ref, a1_ref, w3_ref, hn_ref, hw3_ref):
    h = h_ref[...] + _ssp(a0_ref[...] + a1_ref[...])
    hn_ref[...] = h
    hw3_ref[...] = jnp.dot(h, w3_ref[...], preferred_element_type=jnp.float32)


def _edge_upd_body(e_ref, g_ref, ew4_ref, en_ref):
    en_ref[...] = e_ref[...] + _ssp(g_ref[...] + ew4_ref[...])


def _force_body(e_ref, wo1_ref, bo1_ref, wo2_ref, bo2_ref, uv_ref, fe_ref):
    z1 = _ssp(
        jnp.dot(e_ref[...], wo1_ref[...], preferred_element_type=jnp.float32)
        + bo1_ref[...])
    f = jnp.sum(z1 * wo2_ref[...], axis=1, keepdims=True) + bo2_ref[...]
    fe_ref[...] = f * uv_ref[...]


def _combine_body(p0_ref, p1_ref, o_ref):
    o_ref[...] = p0_ref[...] + p1_ref[...]


_MESH = plsc.VectorSubcoreMesh(
    core_axis_name="c", subcore_axis_name="s", num_cores=_NC, num_subcores=_NS)


def _zero_acc(u, acc, s, ch):
    def _zrow(r, carry):
        for j in range(_F // _L):
            u[r, pl.ds(j * _L, _L)] = jnp.zeros((_L,), jnp.float32)
        return carry
    lax.fori_loop(0, ch, _zrow, 0)

    def _zacc(i, carry):
        pltpu.sync_copy(u, acc.at[pl.ds(pl.multiple_of(s * _RPS + i * ch, 8), ch)])
        return carry
    lax.fori_loop(0, _RPS // ch, _zacc, 0)


def _acc_out(acc, out_hbm, c, s):
    def _wout(i, carry):
        r0 = s * _RPS + i * _CH
        pltpu.sync_copy(acc.at[pl.ds(pl.multiple_of(r0, 8), _CH)],
                        out_hbm.at[pl.ds(pl.multiple_of(c * _NP + r0, 8), _CH)])
        return carry
    lax.fori_loop(0, _RPS // _CH, _wout, 0)


@functools.partial(
    pl.kernel,
    out_type=jax.ShapeDtypeStruct((_NC * _NP, _F), jnp.float32),
    mesh=_MESH,
    scratch_types=[
        pltpu.VMEM((_CHM,), jnp.int32),
        pltpu.VMEM((_CHM,), jnp.int32),
        pltpu.VMEM((_CHM,), jnp.int32),
        pltpu.VMEM((_CHM,), jnp.int32),
        pltpu.VMEM((_CHM, _F), jnp.float32),
        pltpu.VMEM((_CHM, _F), jnp.float32),
        pltpu.VMEM((_CHM, _F), jnp.float32),
        pltpu.VMEM((_CHM, _F), jnp.float32),
        pltpu.VMEM_SHARED((_NP, _F), jnp.float32),
    ] + [pltpu.SemaphoreType.DMA] * 10,
)
def _msg_agg(sw1_hbm, t_hbm, src_hbm, dst_hbm, out_hbm,
             si0, si1, di0, di1, u0, u1, t0, t1, acc,
             smi0, smi1, smd0, smd1, smt0, smt1, smg0, smg1, sms0, sms1):
    c = lax.axis_index("c")
    s = lax.axis_index("s")
    wid = s * _NC + c
    ebase = wid * _EPW
    si = (si0, si1)
    di = (di0, di1)
    u = (u0, u1)
    t = (t0, t1)
    smi = (smi0, smi1)
    smd = (smd0, smd1)
    smt = (smt0, smt1)
    smg = (smg0, smg1)
    sms = (sms0, sms1)

    _zero_acc(u0, acc, s, _CHM)
    plsc.subcore_barrier()

    def drain_scatter(x):
        pltpu.make_async_copy(u[x], acc.at[di[x]], sms[x]).wait()

    def issue(x, g, drain):
        if drain:
            drain_scatter(x)
        base = ebase + g * _CHM
        pltpu.async_copy(src_hbm.at[pl.ds(base, _CHM)], si[x], smi[x])
        pltpu.async_copy(dst_hbm.at[pl.ds(base, _CHM)], di[x], smd[x])
        pltpu.async_copy(t_hbm.at[pl.ds(pl.multiple_of(base, 8), _CHM)], t[x], smt[x])

    def gather(x):
        pltpu.make_async_copy(src_hbm.at[pl.ds(0, _CHM)], si[x], smi[x]).wait()
        pltpu.async_copy(sw1_hbm.at[si[x]], u[x], smg[x])

    def process(x):
        pltpu.make_async_copy(sw1_hbm.at[si[x]], u[x], smg[x]).wait()
        pltpu.make_async_copy(t_hbm.at[pl.ds(0, _CHM)], t[x], smt[x]).wait()

        def _mrow(r2, cc):
            for rr in range(2):
                r = r2 * 2 + rr
                for j in range(_F // _L):
                    sl = pl.ds(j * _L, _L)
                    u[x][r, sl] = u[x][r, sl] * t[x][r, sl]
            return cc
        lax.fori_loop(0, _CHM // 2, _mrow, 0)
        pltpu.make_async_copy(dst_hbm.at[pl.ds(0, _CHM)], di[x], smd[x]).wait()
        pltpu.async_copy(u[x], acc.at[di[x]], sms[x], add=True)

    issue(0, 0, False)
    gather(0)
    issue(1, 1, False)

    def body(i, carry):
        g = 2 * i
        process(0)
        gather(1)
        issue(0, g + 2, True)
        process(1)
        gather(0)
        issue(1, g + 3, True)
        return carry
    lax.fori_loop(0, _NCHUNKM // 2 - 1, body, 0)
    process(0)
    gather(1)
    process(1)
    drain_scatter(0)
    drain_scatter(1)

    plsc.subcore_barrier()
    _acc_out(acc, out_hbm, c, s)


@functools.partial(
    pl.kernel,
    out_type=jax.ShapeDtypeStruct((_EP, _GP), jnp.float32),
    mesh=_MESH,
    scratch_types=[
        pltpu.VMEM((_CH,), jnp.int32),
        pltpu.VMEM((_CH,), jnp.int32),
        pltpu.VMEM((_CH,), jnp.int32),
        pltpu.VMEM((_CH,), jnp.int32),
        pltpu.VMEM((_CH, _F), jnp.float32),
        pltpu.VMEM((_CH, _F), jnp.float32),
        pltpu.VMEM((_CH, _F), jnp.float32),
        pltpu.VMEM((_CH, _F), jnp.float32),
        pltpu.VMEM((_CH, _GP), jnp.float32),
        pltpu.VMEM((_CH, _GP), jnp.float32),
    ] + [pltpu.SemaphoreType.DMA] * 8,
)
def _gather2(hw3_hbm, src_hbm, dst_hbm, out_hbm,
             si0, si1, di0, di1, a0, a1, b0, b1, gn0, gn1,
             smi0, smi1, smd0, smd1, smga0, smga1, smgb0, smgb1):
    c = lax.axis_index("c")
    s = lax.axis_index("s")
    wid = s * _NC + c
    ebase = wid * _EPW
    si = (si0, si1)
    di = (di0, di1)
    a = (a0, a1)
    b = (b0, b1)
    gn = (gn0, gn1)
    smi = (smi0, smi1)
    smd = (smd0, smd1)
    smga = (smga0, smga1)
    smgb = (smgb0, smgb1)

    def issue(x, g):
        base = ebase + g * _CH
        pltpu.async_copy(src_hbm.at[pl.ds(base, _CH)], si[x], smi[x])
        pltpu.async_copy(dst_hbm.at[pl.ds(base, _CH)], di[x], smd[x])

    def gather(x):
        pltpu.make_async_copy(src_hbm.at[pl.ds(0, _CH)], si[x], smi[x]).wait()
        pltpu.async_copy(hw3_hbm.at[si[x]], a[x], smga[x])
        pltpu.make_async_copy(dst_hbm.at[pl.ds(0, _CH)], di[x], smd[x]).wait()
        pltpu.async_copy(hw3_hbm.at[di[x]], b[x], smgb[x])

    def process(x, g):
        base = ebase + g * _CH
        pltpu.make_async_copy(hw3_hbm.at[si[x]], a[x], smga[x]).wait()
        pltpu.make_async_copy(hw3_hbm.at[di[x]], b[x], smgb[x]).wait()

        def _arow(r2, cc):
            for rr in range(2):
                r = r2 * 2 + rr
                for j in range(_GP // _L):
                    sl = pl.ds(j * _L, _L)
                    gn[x][r, sl] = a[x][r, sl] + b[x][r, sl]
            return cc
        lax.fori_loop(0, _CH // 2, _arow, 0)
        pltpu.sync_copy(gn[x], out_hbm.at[pl.ds(pl.multiple_of(base, 8), _CH)])

    issue(0, 0)
    gather(0)
    issue(1, 1)

    def body(i, carry):
        g = 2 * i
        process(0, g)
        gather(1)
        issue(0, g + 2)
        process(1, g + 1)
        gather(0)
        issue(1, g + 3)
        return carry
    lax.fori_loop(0, _NCHUNK // 2 - 1, body, 0)
    process(0, _NCHUNK - 2)
    gather(1)
    process(1, _NCHUNK - 1)


@functools.partial(
    pl.kernel,
    out_type=jax.ShapeDtypeStruct((_NC * _NP, _F), jnp.float32),
    mesh=_MESH,
    scratch_types=[
        pltpu.VMEM((_CH,), jnp.int32),
        pltpu.VMEM((_CH,), jnp.int32),
        pltpu.VMEM((_CH // 8, _F), jnp.float32),
        pltpu.VMEM((_CH // 8, _F), jnp.float32),
        pltpu.VMEM((_CH, _F), jnp.float32),
        pltpu.VMEM((_CH, _F), jnp.float32),
        pltpu.VMEM_SHARED((_NP, _F), jnp.float32),
    ] + [pltpu.SemaphoreType.DMA] * 6,
)
def _scatter_f(fe8_hbm, dst_hbm, out_hbm, di0, di1, fb0, fb1, u0, u1, acc,
               smd0, smd1, smf0, smf1, sms0, sms1):
    c = lax.axis_index("c")
    s = lax.axis_index("s")
    wid = s * _NC + c
    di = (di0, di1)
    fb = (fb0, fb1)
    u = (u0, u1)
    smd = (smd0, smd1)
    smf = (smf0, smf1)
    sms = (sms0, sms1)

    _zero_acc(u0, acc, s, _CH)

    def _zrow(r, carry):
        for j in range(_F // _L):
            u1[r, pl.ds(j * _L, _L)] = jnp.zeros((_L,), jnp.float32)
        return carry
    lax.fori_loop(0, _CH, _zrow, 0)
    plsc.subcore_barrier()

    ebase = wid * _EPW

    def drain_scatter(x):
        pltpu.make_async_copy(u[x], acc.at[di[x]], sms[x]).wait()

    def issue(x, g, drain):
        if drain:
            drain_scatter(x)
        base = ebase + g * _CH
        pltpu.async_copy(dst_hbm.at[pl.ds(base, _CH)], di[x], smd[x])
        pltpu.async_copy(
            fe8_hbm.at[pl.ds(pl.multiple_of(base // 8, 8), _CH // 8)], fb[x], smf[x])

    def process(x):
        pltpu.make_async_copy(fe8_hbm.at[pl.ds(0, _CH // 8)], fb[x], smf[x]).wait()

        def _expand(q, cc):
            for rr in range(8):
                u[x][q * 8 + rr, pl.ds(0, _FW)] = fb[x][q, pl.ds(rr * _FW, _FW)]
            return cc
        lax.fori_loop(0, _CH // 8, _expand, 0)
        pltpu.make_async_copy(dst_hbm.at[pl.ds(0, _CH)], di[x], smd[x]).wait()
        pltpu.async_copy(u[x], acc.at[di[x]], sms[x], add=True)

    issue(0, 0, False)
    issue(1, 1, False)

    def body(i, carry):
        g = 2 * i
        process(0)
        issue(0, g + 2, True)
        process(1)
        issue(1, g + 3, True)
        return carry
    lax.fori_loop(0, _NCHUNK // 2 - 1, body, 0)
    process(0)
    process(1)
    drain_scatter(0)
    drain_scatter(1)

    plsc.subcore_barrier()
    _acc_out(acc, out_hbm, c, s)


_NGRID = _NP // _BN
_EGRID = _EP // _BE


def _full(shape):
    return pl.BlockSpec(shape, lambda i: tuple(0 for _ in shape))


def _rows(shape):
    return pl.BlockSpec(shape, lambda i: (i,) + tuple(0 for _ in shape[1:]))


def _init_nodes(zp, embp):
    return pl.pallas_call(
        _init_nodes_body,
        grid=(_NGRID,),
        in_specs=[_rows((_BN, 1)), _full((128, _F))],
        out_specs=_rows((_BN, _F)),
        out_shape=jax.ShapeDtypeStruct((_NP, _F), jnp.float32),
    )(zp, embp)


def _init_edges(dp, offp, widp):
    return pl.pallas_call(
        _init_edges_body,
        grid=(_EGRID,),
        in_specs=[_rows((_BE, 1)), _full((1, _GP)), _full((1, _GP))],
        out_specs=_rows((_BE, _GP)),
        out_shape=jax.ShapeDtypeStruct((_EP, _GP), jnp.float32),
    )(dp, offp, widp)


def _node_mm(h, w1):
    return pl.pallas_call(
        _node_mm_body,
        grid=(_NGRID,),
        in_specs=[_rows((_BN, _F)), _full((_F, _F))],
        out_specs=_rows((_BN, _F)),
        out_shape=jax.ShapeDtypeStruct((_NP, _F), jnp.float32),
    )(h, w1)


def _edge_mm(e, w2, w4, condp):
    return pl.pallas_call(
        _edge_mm_body,
        grid=(_EGRID,),
        in_specs=[_rows((_BE, _GP)), _full((_GP, _F)), _full((_GP, _GP)),
                  _rows((_BE, 1))],
        out_specs=[_rows((_BE, _F)), _rows((_BE, _GP))],
        out_shape=[jax.ShapeDtypeStruct((_EP, _F), jnp.float32),
                   jax.ShapeDtypeStruct((_EP, _GP), jnp.float32)],
    )(e, w2, w4, condp)


def _node_upd(h, aggp, w3):
    a0 = pl.BlockSpec((_BN, _F), lambda i: (i, 0))
    a1 = pl.BlockSpec((_BN, _F), lambda i: (i + _NGRID, 0))
    return pl.pallas_call(
        _node_upd_body,
        grid=(_NGRID,),
        in_specs=[_rows((_BN, _F)), a0, a1, _full((_F, _F))],
        out_specs=[_rows((_BN, _F)), _rows((_BN, _F))],
        out_shape=[jax.ShapeDtypeStruct((_NP, _F), jnp.float32),
                   jax.ShapeDtypeStruct((_NP, _F), jnp.float32)],
    )(h, aggp, aggp, w3)


def _edge_upd(e, g, ew4):
    return pl.pallas_call(
        _edge_upd_body,
        grid=(_EGRID,),
        in_specs=[_rows((_BE, _GP))] * 3,
        out_specs=_rows((_BE, _GP)),
        out_shape=jax.ShapeDtypeStruct((_EP, _GP), jnp.float32),
    )(e, g, ew4)


def _force(e, wo1, bo1, wo2t, bo2, uvp):
    return pl.pallas_call(
        _force_body,
        grid=(_EGRID,),
        in_specs=[_rows((_BE, _GP)), _full((_GP, _GP)), _full((1, _GP)),
                  _full((1, _GP)), _full((1, 1)), _rows((_BE, _FW))],
        out_specs=_rows((_BE, _FW)),
        out_shape=jax.ShapeDtypeStruct((_EP, _FW), jnp.float32),
    )(e, wo1, bo1, wo2t, bo2, uvp)


def _combine(fp):
    p0 = pl.BlockSpec((_BN, _F), lambda i: (i, 0))
    p1 = pl.BlockSpec((_BN, _F), lambda i: (i + _NGRID, 0))
    return pl.pallas_call(
        _combine_body,
        grid=(_NGRID,),
        in_specs=[p0, p1],
        out_specs=_rows((_BN, _F)),
        out_shape=jax.ShapeDtypeStruct((_NP, _F), jnp.float32),
    )(fp, fp)


def kernel(Z, edge_index, distances, unit_vecs, conductance, emb_table,
           g_offsets, g_widths, W1, W2, W3, W4, Wo1, bo1, Wo2, bo2):
    f32 = jnp.float32
    epad = _EP - _E
    npad = _NP - _N

    src = jnp.pad(edge_index[0].astype(jnp.int32), (0, epad))
    dst = jnp.pad(edge_index[1].astype(jnp.int32), (0, epad))
    dp = jnp.pad(distances.astype(f32), (0, epad),
                 constant_values=1.0).reshape(_EP, 1)
    condp = jnp.pad(conductance.astype(f32), (0, epad)).reshape(_EP, 1)
    uvp = jnp.pad(unit_vecs.astype(f32), ((0, epad), (0, _FW - 3)))
    zp = jnp.pad(Z.astype(jnp.int32), (0, npad)).reshape(_NP, 1)
    embp = jnp.pad(emb_table.astype(f32), ((0, 128 - emb_table.shape[0]), (0, 0)))
    offp = jnp.pad(g_offsets.astype(f32), (0, _GP - _G)).reshape(1, _GP)
    widp = jnp.pad(g_widths.astype(f32), (0, _GP - _G),
                   constant_values=1.0).reshape(1, _GP)
    W1 = W1.astype(f32)
    W2p = jnp.pad(W2.astype(f32), ((0, 0), (0, _GP - _G), (0, 0)))
    W3p = jnp.pad(W3.astype(f32), ((0, 0), (0, 0), (0, _F - _G)))
    W4p = jnp.pad(W4.astype(f32), ((0, 0), (0, _GP - _G), (0, _GP - _G)))
    Wo1p = jnp.pad(Wo1.astype(f32), ((0, _GP - _G), (0, _GP - _G)))
    bo1p = jnp.pad(bo1.astype(f32), (0, _GP - _G)).reshape(1, _GP)
    Wo2t = jnp.pad(Wo2.astype(f32), ((0, _GP - _G), (0, 0))).reshape(1, _GP)
    bo2p = bo2.astype(f32).reshape(1, 1)

    h = _init_nodes(zp, embp)
    e = _init_edges(dp, offp, widp)

    for l in range(_NMP):
        sw1 = _node_mm(h, W1[l])
        t, ew4 = _edge_mm(e, W2p[l], W4p[l], condp)
        aggp = _msg_agg(sw1, t, src, dst)
        h, hw3 = _node_upd(h, aggp, W3p[l])
        g = _gather2(hw3, src, dst)
        e = _edge_upd(e, g, ew4)

    fe = _force(e, Wo1p, bo1p, Wo2t, bo2p, uvp)
    fp = _scatter_f(fe.reshape(_EP // 8, _F), dst)
    forces = _combine(fp)
    return forces[:_N, :3]


# speedup vs baseline: 1.6430x; 1.0493x over previous
import functools

import jax
import jax.numpy as jnp
from jax import lax
from jax.experimental import pallas as pl
from jax.experimental.pallas import tpu as pltpu
from jax.experimental.pallas import tpu_sc as plsc

_N = 10000
_E = 320000
_F = 128
_G = 20
_GP = 32
_NMP = 3
_NC = 2
_NS = 16
_L = 16
_NW = _NC * _NS
_NP = 10240
_EP = 327680
_CH = 128
_CHM = 64
_EPW = _EP // _NW
_NCHUNK = _EPW // _CH
_NCHUNKM = _EPW // _CHM
_RPS = _NP // _NS
_FW = 16
_LN2 = 0.6931471805599453
_BN = 512
_BE = 2048


def _ssp(x):
    return jnp.maximum(x, 0.0) + jnp.log1p(jnp.exp(-jnp.abs(x))) - _LN2


def _init_nodes_body(z_ref, emb_ref, h_ref):
    z = z_ref[...]
    oh = (z == lax.broadcasted_iota(jnp.int32, (_BN, 128), 1)).astype(jnp.float32)
    h_ref[...] = jnp.dot(oh, emb_ref[...], preferred_element_type=jnp.float32)


def _init_edges_body(d_ref, off_ref, w_ref, e_ref):
    d = d_ref[...]
    off = off_ref[...]
    w = w_ref[...]
    e = jnp.exp(-((d - off) ** 2) / (2.0 * w * w))
    mask = lax.broadcasted_iota(jnp.int32, e.shape, 1) < _G
    e_ref[...] = jnp.where(mask, e, 0.0)


def _node_mm_body(h_ref, w1_ref, sw1_ref):
    sw1_ref[...] = _ssp(
        jnp.dot(h_ref[...], w1_ref[...], preferred_element_type=jnp.float32))


def _edge_mm_body(e_ref, w2_ref, w4_ref, c_ref, t_ref, ew4_ref):
    e = e_ref[...]
    t_ref[...] = _ssp(
        jnp.dot(e, w2_ref[...], preferred_element_type=jnp.float32)) * c_ref[...]
    ew4_ref[...] = jnp.dot(e, w4_ref[...], preferred_element_type=jnp.float32)


def _node_upd_body(h_ref, a0_ref, a1_ref, w3_ref, hn_ref, hw3_ref):
    h = h_ref[...] + _ssp(a0_ref[...] + a1_ref[...])
    hn_ref[...] = h
    hw3_ref[...] = jnp.dot(h, w3_ref[...], preferred_element_type=jnp.float32)


def _edge_upd_body(e_ref, g_ref, ew4_ref, en_ref):
    en_ref[...] = e_ref[...] + _ssp(g_ref[...] + ew4_ref[...])


def _force_body(e_ref, wo1_ref, bo1_ref, wo2_ref, bo2_ref, uv_ref, fe_ref):
    z1 = _ssp(
        jnp.dot(e_ref[...], wo1_ref[...], preferred_element_type=jnp.float32)
        + bo1_ref[...])
    f = jnp.sum(z1 * wo2_ref[...], axis=1, keepdims=True) + bo2_ref[...]
    fe_ref[...] = f * uv_ref[...]


def _combine_body(p0_ref, p1_ref, o_ref):
    o_ref[...] = p0_ref[...] + p1_ref[...]


_MESH = plsc.VectorSubcoreMesh(
    core_axis_name="c", subcore_axis_name="s", num_cores=_NC, num_subcores=_NS)


def _zero_acc(u, acc, s, ch):
    def _zrow(r, carry):
        for j in range(_F // _L):
            u[r, pl.ds(j * _L, _L)] = jnp.zeros((_L,), jnp.float32)
        return carry
    lax.fori_loop(0, ch, _zrow, 0)

    def _zacc(i, carry):
        pltpu.sync_copy(u, acc.at[pl.ds(pl.multiple_of(s * _RPS + i * ch, 8), ch)])
        return carry
    lax.fori_loop(0, _RPS // ch, _zacc, 0)


def _acc_out(acc, out_hbm, c, s):
    def _wout(i, carry):
        r0 = s * _RPS + i * _CH
        pltpu.sync_copy(acc.at[pl.ds(pl.multiple_of(r0, 8), _CH)],
                        out_hbm.at[pl.ds(pl.multiple_of(c * _NP + r0, 8), _CH)])
        return carry
    lax.fori_loop(0, _RPS // _CH, _wout, 0)


@functools.partial(
    pl.kernel,
    out_type=jax.ShapeDtypeStruct((_NC * _NP, _F), jnp.float32),
    mesh=_MESH,
    scratch_types=[
        pltpu.VMEM((_CHM,), jnp.int32),
        pltpu.VMEM((_CHM,), jnp.int32),
        pltpu.VMEM((_CHM,), jnp.int32),
        pltpu.VMEM((_CHM,), jnp.int32),
        pltpu.VMEM((_CHM, _F), jnp.float32),
        pltpu.VMEM((_CHM, _F), jnp.float32),
        pltpu.VMEM((_CHM, _F), jnp.float32),
        pltpu.VMEM((_CHM, _F), jnp.float32),
        pltpu.VMEM_SHARED((_NP, _F), jnp.float32),
    ] + [pltpu.SemaphoreType.DMA] * 10,
)
def _msg_agg(sw1_hbm, t_hbm, src_hbm, dst_hbm, out_hbm,
             si0, si1, di0, di1, u0, u1, t0, t1, acc,
             smi0, smi1, smd0, smd1, smt0, smt1, smg0, smg1, sms0, sms1):
    c = lax.axis_index("c")
    s = lax.axis_index("s")
    wid = s * _NC + c
    ebase = wid * _EPW
    si = (si0, si1)
    di = (di0, di1)
    u = (u0, u1)
    t = (t0, t1)
    smi = (smi0, smi1)
    smd = (smd0, smd1)
    smt = (smt0, smt1)
    smg = (smg0, smg1)
    sms = (sms0, sms1)

    _zero_acc(u0, acc, s, _CHM)
    plsc.subcore_barrier()

    def drain_scatter(x):
        pltpu.make_async_copy(u[x], acc.at[di[x]], sms[x]).wait()

    def issue(x, g, drain):
        if drain:
            drain_scatter(x)
        base = ebase + g * _CHM
        pltpu.async_copy(src_hbm.at[pl.ds(base, _CHM)], si[x], smi[x])
        pltpu.async_copy(dst_hbm.at[pl.ds(base, _CHM)], di[x], smd[x])
        pltpu.async_copy(t_hbm.at[pl.ds(pl.multiple_of(base, 8), _CHM)], t[x], smt[x])

    def gather(x):
        pltpu.make_async_copy(src_hbm.at[pl.ds(0, _CHM)], si[x], smi[x]).wait()
        pltpu.async_copy(sw1_hbm.at[si[x]], u[x], smg[x])

    def process(x):
        pltpu.make_async_copy(sw1_hbm.at[si[x]], u[x], smg[x]).wait()
        pltpu.make_async_copy(t_hbm.at[pl.ds(0, _CHM)], t[x], smt[x]).wait()

        def _mrow(r2, cc):
            for rr in range(2):
                r = r2 * 2 + rr
                for j in range(_F // _L):
                    sl = pl.ds(j * _L, _L)
                    u[x][r, sl] = u[x][r, sl] * t[x][r, sl]
            return cc
        lax.fori_loop(0, _CHM // 2, _mrow, 0)
        pltpu.make_async_copy(dst_hbm.at[pl.ds(0, _CHM)], di[x], smd[x]).wait()
        pltpu.async_copy(u[x], acc.at[di[x]], sms[x], add=True)

    issue(0, 0, False)
    gather(0)
    issue(1, 1, False)

    def body(i, carry):
        g = 2 * i
        process(0)
        gather(1)
        issue(0, g + 2, True)
        process(1)
        gather(0)
        issue(1, g + 3, True)
        return carry
    lax.fori_loop(0, _NCHUNKM // 2 - 1, body, 0)
    process(0)
    gather(1)
    process(1)
    drain_scatter(0)
    drain_scatter(1)

    plsc.subcore_barrier()
    _acc_out(acc, out_hbm, c, s)


@functools.partial(
    pl.kernel,
    out_type=jax.ShapeDtypeStruct((_EP, _GP), jnp.float32),
    mesh=_MESH,
    scratch_types=[
        pltpu.VMEM((_CH,), jnp.int32),
        pltpu.VMEM((_CH,), jnp.int32),
        pltpu.VMEM((_CH,), jnp.int32),
        pltpu.VMEM((_CH,), jnp.int32),
        pltpu.VMEM((_CH,), jnp.int32),
        pltpu.VMEM((_CH,), jnp.int32),
        pltpu.VMEM((_CH, _F), jnp.float32),
        pltpu.VMEM((_CH, _F), jnp.float32),
        pltpu.VMEM((_CH, _F), jnp.float32),
        pltpu.VMEM((_CH, _F), jnp.float32),
        pltpu.VMEM((_CH, _F), jnp.float32),
        pltpu.VMEM((_CH, _F), jnp.float32),
        pltpu.VMEM((_CH, _GP), jnp.float32),
    ] + [pltpu.SemaphoreType.DMA] * 12,
)
def _gather2(hw3_hbm, src_hbm, dst_hbm, out_hbm,
             si0, si1, si2, di0, di1, di2, a0, a1, a2, b0, b1, b2, gn,
             smi0, smi1, smi2, smd0, smd1, smd2,
             smga0, smga1, smga2, smgb0, smgb1, smgb2):
    c = lax.axis_index("c")
    s = lax.axis_index("s")
    wid = s * _NC + c
    ebase = wid * _EPW
    si = (si0, si1, si2)
    di = (di0, di1, di2)
    a = (a0, a1, a2)
    b = (b0, b1, b2)
    smi = (smi0, smi1, smi2)
    smd = (smd0, smd1, smd2)
    smga = (smga0, smga1, smga2)
    smgb = (smgb0, smgb1, smgb2)

    def issue(x, g):
        base = ebase + g * _CH
        pltpu.async_copy(src_hbm.at[pl.ds(base, _CH)], si[x], smi[x])
        pltpu.async_copy(dst_hbm.at[pl.ds(base, _CH)], di[x], smd[x])

    def gather(x):
        pltpu.make_async_copy(src_hbm.at[pl.ds(0, _CH)], si[x], smi[x]).wait()
        pltpu.async_copy(hw3_hbm.at[si[x]], a[x], smga[x])
        pltpu.make_async_copy(dst_hbm.at[pl.ds(0, _CH)], di[x], smd[x]).wait()
        pltpu.async_copy(hw3_hbm.at[di[x]], b[x], smgb[x])

    def process(x, g):
        base = ebase + g * _CH
        pltpu.make_async_copy(hw3_hbm.at[si[x]], a[x], smga[x]).wait()
        pltpu.make_async_copy(hw3_hbm.at[di[x]], b[x], smgb[x]).wait()

        def _arow(r2, cc):
            for rr in range(2):
                r = r2 * 2 + rr
                for j in range(_GP // _L):
                    sl = pl.ds(j * _L, _L)
                    gn[r, sl] = a[x][r, sl] + b[x][r, sl]
            return cc
        lax.fori_loop(0, _CH // 2, _arow, 0)
        pltpu.sync_copy(gn, out_hbm.at[pl.ds(pl.multiple_of(base, 8), _CH)])

    issue(0, 0)
    gather(0)
    issue(1, 1)
    gather(1)
    issue(2, 2)
    gather(2)

    def body(i, carry):
        g = 3 * i
        process(0, g)
        issue(0, g + 3)
        gather(0)
        process(1, g + 1)
        issue(1, g + 4)
        gather(1)
        process(2, g + 2)
        issue(2, g + 5)
        gather(2)
        return carry
    lax.fori_loop(0, (_NCHUNK - 5) // 3, body, 0)
    process(0, _NCHUNK - 5)
    issue(0, _NCHUNK - 2)
    gather(0)
    process(1, _NCHUNK - 4)
    issue(1, _NCHUNK - 1)
    gather(1)
    process(2, _NCHUNK - 3)
    process(0, _NCHUNK - 2)
    process(1, _NCHUNK - 1)


@functools.partial(
    pl.kernel,
    out_type=jax.ShapeDtypeStruct((_NC * _NP, _F), jnp.float32),
    mesh=_MESH,
    scratch_types=[
        pltpu.VMEM((_CH,), jnp.int32),
        pltpu.VMEM((_CH,), jnp.int32),
        pltpu.VMEM((_CH // 8, _F), jnp.float32),
        pltpu.VMEM((_CH // 8, _F), jnp.float32),
        pltpu.VMEM((_CH, _F), jnp.float32),
        pltpu.VMEM((_CH, _F), jnp.float32),
        pltpu.VMEM_SHARED((_NP, _F), jnp.float32),
    ] + [pltpu.SemaphoreType.DMA] * 6,
)
def _scatter_f(fe8_hbm, dst_hbm, out_hbm, di0, di1, fb0, fb1, u0, u1, acc,
               smd0, smd1, smf0, smf1, sms0, sms1):
    c = lax.axis_index("c")
    s = lax.axis_index("s")
    wid = s * _NC + c
    di = (di0, di1)
    fb = (fb0, fb1)
    u = (u0, u1)
    smd = (smd0, smd1)
    smf = (smf0, smf1)
    sms = (sms0, sms1)

    _zero_acc(u0, acc, s, _CH)

    def _zrow(r, carry):
        for j in range(_F // _L):
            u1[r, pl.ds(j * _L, _L)] = jnp.zeros((_L,), jnp.float32)
        return carry
    lax.fori_loop(0, _CH, _zrow, 0)
    plsc.subcore_barrier()

    ebase = wid * _EPW

    def drain_scatter(x):
        pltpu.make_async_copy(u[x], acc.at[di[x]], sms[x]).wait()

    def issue(x, g, drain):
        if drain:
            drain_scatter(x)
        base = ebase + g * _CH
        pltpu.async_copy(dst_hbm.at[pl.ds(base, _CH)], di[x], smd[x])
        pltpu.async_copy(
            fe8_hbm.at[pl.ds(pl.multiple_of(base // 8, 8), _CH // 8)], fb[x], smf[x])

    def process(x):
        pltpu.make_async_copy(fe8_hbm.at[pl.ds(0, _CH // 8)], fb[x], smf[x]).wait()

        def _expand(q, cc):
            for rr in range(8):
                u[x][q * 8 + rr, pl.ds(0, _FW)] = fb[x][q, pl.ds(rr * _FW, _FW)]
            return cc
        lax.fori_loop(0, _CH // 8, _expand, 0)
        pltpu.make_async_copy(dst_hbm.at[pl.ds(0, _CH)], di[x], smd[x]).wait()
        pltpu.async_copy(u[x], acc.at[di[x]], sms[x], add=True)

    issue(0, 0, False)
    issue(1, 1, False)

    def body(i, carry):
        g = 2 * i
        process(0)
        issue(0, g + 2, True)
        process(1)
        issue(1, g + 3, True)
        return carry
    lax.fori_loop(0, _NCHUNK // 2 - 1, body, 0)
    process(0)
    process(1)
    drain_scatter(0)
    drain_scatter(1)

    plsc.subcore_barrier()
    _acc_out(acc, out_hbm, c, s)


_NGRID = _NP // _BN
_EGRID = _EP // _BE


def _full(shape):
    return pl.BlockSpec(shape, lambda i: tuple(0 for _ in shape))


def _rows(shape):
    return pl.BlockSpec(shape, lambda i: (i,) + tuple(0 for _ in shape[1:]))


def _init_nodes(zp, embp):
    return pl.pallas_call(
        _init_nodes_body,
        grid=(_NGRID,),
        in_specs=[_rows((_BN, 1)), _full((128, _F))],
        out_specs=_rows((_BN, _F)),
        out_shape=jax.ShapeDtypeStruct((_NP, _F), jnp.float32),
    )(zp, embp)


def _init_edges(dp, offp, widp):
    return pl.pallas_call(
        _init_edges_body,
        grid=(_EGRID,),
        in_specs=[_rows((_BE, 1)), _full((1, _GP)), _full((1, _GP))],
        out_specs=_rows((_BE, _GP)),
        out_shape=jax.ShapeDtypeStruct((_EP, _GP), jnp.float32),
    )(dp, offp, widp)


def _node_mm(h, w1):
    return pl.pallas_call(
        _node_mm_body,
        grid=(_NGRID,),
        in_specs=[_rows((_BN, _F)), _full((_F, _F))],
        out_specs=_rows((_BN, _F)),
        out_shape=jax.ShapeDtypeStruct((_NP, _F), jnp.float32),
    )(h, w1)


def _edge_mm(e, w2, w4, condp):
    return pl.pallas_call(
        _edge_mm_body,
        grid=(_EGRID,),
        in_specs=[_rows((_BE, _GP)), _full((_GP, _F)), _full((_GP, _GP)),
                  _rows((_BE, 1))],
        out_specs=[_rows((_BE, _F)), _rows((_BE, _GP))],
        out_shape=[jax.ShapeDtypeStruct((_EP, _F), jnp.float32),
                   jax.ShapeDtypeStruct((_EP, _GP), jnp.float32)],
    )(e, w2, w4, condp)


def _node_upd(h, aggp, w3):
    a0 = pl.BlockSpec((_BN, _F), lambda i: (i, 0))
    a1 = pl.BlockSpec((_BN, _F), lambda i: (i + _NGRID, 0))
    return pl.pallas_call(
        _node_upd_body,
        grid=(_NGRID,),
        in_specs=[_rows((_BN, _F)), a0, a1, _full((_F, _F))],
        out_specs=[_rows((_BN, _F)), _rows((_BN, _F))],
        out_shape=[jax.ShapeDtypeStruct((_NP, _F), jnp.float32),
                   jax.ShapeDtypeStruct((_NP, _F), jnp.float32)],
    )(h, aggp, aggp, w3)


def _edge_upd(e, g, ew4):
    return pl.pallas_call(
        _edge_upd_body,
        grid=(_EGRID,),
        in_specs=[_rows((_BE, _GP))] * 3,
        out_specs=_rows((_BE, _GP)),
        out_shape=jax.ShapeDtypeStruct((_EP, _GP), jnp.float32),
    )(e, g, ew4)


def _force(e, wo1, bo1, wo2t, bo2, uvp):
    return pl.pallas_call(
        _force_body,
        grid=(_EGRID,),
        in_specs=[_rows((_BE, _GP)), _full((_GP, _GP)), _full((1, _GP)),
                  _full((1, _GP)), _full((1, 1)), _rows((_BE, _FW))],
        out_specs=_rows((_BE, _FW)),
        out_shape=jax.ShapeDtypeStruct((_EP, _FW), jnp.float32),
    )(e, wo1, bo1, wo2t, bo2, uvp)


def _combine(fp):
    p0 = pl.BlockSpec((_BN, _F), lambda i: (i, 0))
    p1 = pl.BlockSpec((_BN, _F), lambda i: (i + _NGRID, 0))
    return pl.pallas_call(
        _combine_body,
        grid=(_NGRID,),
        in_specs=[p0, p1],
        out_specs=_rows((_BN, _F)),
        out_shape=jax.ShapeDtypeStruct((_NP, _F), jnp.float32),
    )(fp, fp)


def kernel(Z, edge_index, distances, unit_vecs, conductance, emb_table,
           g_offsets, g_widths, W1, W2, W3, W4, Wo1, bo1, Wo2, bo2):
    f32 = jnp.float32
    epad = _EP - _E
    npad = _NP - _N

    src = jnp.pad(edge_index[0].astype(jnp.int32), (0, epad))
    dst = jnp.pad(edge_index[1].astype(jnp.int32), (0, epad))
    dp = jnp.pad(distances.astype(f32), (0, epad),
                 constant_values=1.0).reshape(_EP, 1)
    condp = jnp.pad(conductance.astype(f32), (0, epad)).reshape(_EP, 1)
    uvp = jnp.pad(unit_vecs.astype(f32), ((0, epad), (0, _FW - 3)))
    zp = jnp.pad(Z.astype(jnp.int32), (0, npad)).reshape(_NP, 1)
    embp = jnp.pad(emb_table.astype(f32), ((0, 128 - emb_table.shape[0]), (0, 0)))
    offp = jnp.pad(g_offsets.astype(f32), (0, _GP - _G)).reshape(1, _GP)
    widp = jnp.pad(g_widths.astype(f32), (0, _GP - _G),
                   constant_values=1.0).reshape(1, _GP)
    W1 = W1.astype(f32)
    W2p = jnp.pad(W2.astype(f32), ((0, 0), (0, _GP - _G), (0, 0)))
    W3p = jnp.pad(W3.astype(f32), ((0, 0), (0, 0), (0, _F - _G)))
    W4p = jnp.pad(W4.astype(f32), ((0, 0), (0, _GP - _G), (0, _GP - _G)))
    Wo1p = jnp.pad(Wo1.astype(f32), ((0, _GP - _G), (0, _GP - _G)))
    bo1p = jnp.pad(bo1.astype(f32), (0, _GP - _G)).reshape(1, _GP)
    Wo2t = jnp.pad(Wo2.astype(f32), ((0, _GP - _G), (0, 0))).reshape(1, _GP)
    bo2p = bo2.astype(f32).reshape(1, 1)

    h = _init_nodes(zp, embp)
    e = _init_edges(dp, offp, widp)

    for l in range(_NMP):
        sw1 = _node_mm(h, W1[l])
        t, ew4 = _edge_mm(e, W2p[l], W4p[l], condp)
        aggp = _msg_agg(sw1, t, src, dst)
        h, hw3 = _node_upd(h, aggp, W3p[l])
        g = _gather2(hw3, src, dst)
        e = _edge_upd(e, g, ew4)

    fe = _force(e, Wo1p, bo1p, Wo2t, bo2p, uvp)
    fp = _scatter_f(fe.reshape(_EP // 8, _F), dst)
    forces = _combine(fp)
    return forces[:_N, :3]


# speedup vs baseline: 1.6557x; 1.0077x over previous
import functools

import jax
import jax.numpy as jnp
from jax import lax
from jax.experimental import pallas as pl
from jax.experimental.pallas import tpu as pltpu
from jax.experimental.pallas import tpu_sc as plsc

_N = 10000
_E = 320000
_F = 128
_G = 20
_GP = 32
_NMP = 3
_NC = 2
_NS = 16
_L = 16
_NW = _NC * _NS
_NP = 10240
_EP = 327680
_CH = 128
_CHM = 80
_EPW = _EP // _NW
_NCHUNK = _EPW // _CH
_NCHUNKM = _EPW // _CHM
_RPS = _NP // _NS
_FW = 16
_LN2 = 0.6931471805599453
_BN = 512
_BE = 2048


def _ssp(x):
    return jnp.maximum(x, 0.0) + jnp.log1p(jnp.exp(-jnp.abs(x))) - _LN2


def _init_nodes_body(z_ref, emb_ref, h_ref):
    z = z_ref[...]
    oh = (z == lax.broadcasted_iota(jnp.int32, (_BN, 128), 1)).astype(jnp.float32)
    h_ref[...] = jnp.dot(oh, emb_ref[...], preferred_element_type=jnp.float32)


def _init_edges_body(d_ref, off_ref, w_ref, e_ref):
    d = d_ref[...]
    off = off_ref[...]
    w = w_ref[...]
    e = jnp.exp(-((d - off) ** 2) / (2.0 * w * w))
    mask = lax.broadcasted_iota(jnp.int32, e.shape, 1) < _G
    e_ref[...] = jnp.where(mask, e, 0.0)


def _node_mm_body(h_ref, w1_ref, sw1_ref):
    sw1_ref[...] = _ssp(
        jnp.dot(h_ref[...], w1_ref[...], preferred_element_type=jnp.float32))


def _edge_mm_body(e_ref, w2_ref, w4_ref, c_ref, t_ref, ew4_ref):
    e = e_ref[...]
    t_ref[...] = _ssp(
        jnp.dot(e, w2_ref[...], preferred_element_type=jnp.float32)) * c_ref[...]
    ew4_ref[...] = jnp.dot(e, w4_ref[...], preferred_element_type=jnp.float32)


def _node_upd_body(h_ref, a0_ref, a1_ref, w3_ref, hn_ref, hw3_ref):
    h = h_ref[...] + _ssp(a0_ref[...] + a1_ref[...])
    hn_ref[...] = h
    hw3_ref[...] = jnp.dot(h, w3_ref[...], preferred_element_type=jnp.float32)


def _edge_upd_body(e_ref, g_ref, ew4_ref, en_ref):
    en_ref[...] = e_ref[...] + _ssp(g_ref[...] + ew4_ref[...])


def _force_body(e_ref, wo1_ref, bo1_ref, wo2_ref, bo2_ref, uv_ref, fe_ref):
    z1 = _ssp(
        jnp.dot(e_ref[...], wo1_ref[...], preferred_element_type=jnp.float32)
        + bo1_ref[...])
    f = jnp.sum(z1 * wo2_ref[...], axis=1, keepdims=True) + bo2_ref[...]
    fe_ref[...] = f * uv_ref[...]


def _combine_body(p0_ref, p1_ref, o_ref):
    o_ref[...] = p0_ref[...] + p1_ref[...]


_MESH = plsc.VectorSubcoreMesh(
    core_axis_name="c", subcore_axis_name="s", num_cores=_NC, num_subcores=_NS)


def _zero_acc(u, acc, s, ch):
    def _zrow(r, carry):
        for j in range(_F // _L):
            u[r, pl.ds(j * _L, _L)] = jnp.zeros((_L,), jnp.float32)
        return carry
    lax.fori_loop(0, ch, _zrow, 0)

    def _zacc(i, carry):
        pltpu.sync_copy(u, acc.at[pl.ds(pl.multiple_of(s * _RPS + i * ch, 8), ch)])
        return carry
    lax.fori_loop(0, _RPS // ch, _zacc, 0)


def _acc_out(acc, out_hbm, c, s):
    def _wout(i, carry):
        r0 = s * _RPS + i * _CH
        pltpu.sync_copy(acc.at[pl.ds(pl.multiple_of(r0, 8), _CH)],
                        out_hbm.at[pl.ds(pl.multiple_of(c * _NP + r0, 8), _CH)])
        return carry
    lax.fori_loop(0, _RPS // _CH, _wout, 0)


@functools.partial(
    pl.kernel,
    out_type=jax.ShapeDtypeStruct((_NC * _NP, _F), jnp.float32),
    mesh=_MESH,
    scratch_types=[
        pltpu.VMEM((_CHM,), jnp.int32),
        pltpu.VMEM((_CHM,), jnp.int32),
        pltpu.VMEM((_CHM,), jnp.int32),
        pltpu.VMEM((_CHM,), jnp.int32),
        pltpu.VMEM((_CHM, _F), jnp.float32),
        pltpu.VMEM((_CHM, _F), jnp.float32),
        pltpu.VMEM((_CHM, _F), jnp.float32),
        pltpu.VMEM((_CHM, _F), jnp.float32),
        pltpu.VMEM_SHARED((_NP, _F), jnp.float32),
    ] + [pltpu.SemaphoreType.DMA] * 10,
)
def _msg_agg(sw1_hbm, t_hbm, src_hbm, dst_hbm, out_hbm,
             si0, si1, di0, di1, u0, u1, t0, t1, acc,
             smi0, smi1, smd0, smd1, smt0, smt1, smg0, smg1, sms0, sms1):
    c = lax.axis_index("c")
    s = lax.axis_index("s")
    wid = s * _NC + c
    ebase = wid * _EPW
    si = (si0, si1)
    di = (di0, di1)
    u = (u0, u1)
    t = (t0, t1)
    smi = (smi0, smi1)
    smd = (smd0, smd1)
    smt = (smt0, smt1)
    smg = (smg0, smg1)
    sms = (sms0, sms1)

    _zero_acc(u0, acc, s, _CHM)
    plsc.subcore_barrier()

    def drain_scatter(x):
        pltpu.make_async_copy(u[x], acc.at[di[x]], sms[x]).wait()

    def issue(x, g, drain):
        if drain:
            drain_scatter(x)
        base = ebase + g * _CHM
        pltpu.async_copy(src_hbm.at[pl.ds(base, _CHM)], si[x], smi[x])
        pltpu.async_copy(dst_hbm.at[pl.ds(base, _CHM)], di[x], smd[x])
        pltpu.async_copy(t_hbm.at[pl.ds(pl.multiple_of(base, 8), _CHM)], t[x], smt[x])

    def gather(x):
        pltpu.make_async_copy(src_hbm.at[pl.ds(0, _CHM)], si[x], smi[x]).wait()
        pltpu.async_copy(sw1_hbm.at[si[x]], u[x], smg[x])

    def process(x):
        pltpu.make_async_copy(sw1_hbm.at[si[x]], u[x], smg[x]).wait()
        pltpu.make_async_copy(t_hbm.at[pl.ds(0, _CHM)], t[x], smt[x]).wait()

        def _mrow(r2, cc):
            for rr in range(2):
                r = r2 * 2 + rr
                for j in range(_F // _L):
                    sl = pl.ds(j * _L, _L)
                    u[x][r, sl] = u[x][r, sl] * t[x][r, sl]
            return cc
        lax.fori_loop(0, _CHM // 2, _mrow, 0)
        pltpu.make_async_copy(dst_hbm.at[pl.ds(0, _CHM)], di[x], smd[x]).wait()
        pltpu.async_copy(u[x], acc.at[di[x]], sms[x], add=True)

    issue(0, 0, False)
    gather(0)
    issue(1, 1, False)

    def body(i, carry):
        g = 2 * i
        process(0)
        gather(1)
        issue(0, g + 2, True)
        process(1)
        gather(0)
        issue(1, g + 3, True)
        return carry
    lax.fori_loop(0, _NCHUNKM // 2 - 1, body, 0)
    process(0)
    gather(1)
    process(1)
    drain_scatter(0)
    drain_scatter(1)

    plsc.subcore_barrier()
    _acc_out(acc, out_hbm, c, s)


@functools.partial(
    pl.kernel,
    out_type=jax.ShapeDtypeStruct((_EP, _GP), jnp.float32),
    mesh=_MESH,
    scratch_types=[
        pltpu.VMEM((_CH,), jnp.int32),
        pltpu.VMEM((_CH,), jnp.int32),
        pltpu.VMEM((_CH,), jnp.int32),
        pltpu.VMEM((_CH,), jnp.int32),
        pltpu.VMEM((_CH,), jnp.int32),
        pltpu.VMEM((_CH,), jnp.int32),
        pltpu.VMEM((_CH, _F), jnp.float32),
        pltpu.VMEM((_CH, _F), jnp.float32),
        pltpu.VMEM((_CH, _F), jnp.float32),
        pltpu.VMEM((_CH, _F), jnp.float32),
        pltpu.VMEM((_CH, _F), jnp.float32),
        pltpu.VMEM((_CH, _F), jnp.float32),
        pltpu.VMEM((_CH, _GP), jnp.float32),
    ] + [pltpu.SemaphoreType.DMA] * 12,
)
def _gather2(hw3_hbm, src_hbm, dst_hbm, out_hbm,
             si0, si1, si2, di0, di1, di2, a0, a1, a2, b0, b1, b2, gn,
             smi0, smi1, smi2, smd0, smd1, smd2,
             smga0, smga1, smga2, smgb0, smgb1, smgb2):
    c = lax.axis_index("c")
    s = lax.axis_index("s")
    wid = s * _NC + c
    ebase = wid * _EPW
    si = (si0, si1, si2)
    di = (di0, di1, di2)
    a = (a0, a1, a2)
    b = (b0, b1, b2)
    smi = (smi0, smi1, smi2)
    smd = (smd0, smd1, smd2)
    smga = (smga0, smga1, smga2)
    smgb = (smgb0, smgb1, smgb2)

    def issue(x, g):
        base = ebase + g * _CH
        pltpu.async_copy(src_hbm.at[pl.ds(base, _CH)], si[x], smi[x])
        pltpu.async_copy(dst_hbm.at[pl.ds(base, _CH)], di[x], smd[x])

    def gather(x):
        pltpu.make_async_copy(src_hbm.at[pl.ds(0, _CH)], si[x], smi[x]).wait()
        pltpu.async_copy(hw3_hbm.at[si[x]], a[x], smga[x])
        pltpu.make_async_copy(dst_hbm.at[pl.ds(0, _CH)], di[x], smd[x]).wait()
        pltpu.async_copy(hw3_hbm.at[di[x]], b[x], smgb[x])

    def process(x, g):
        base = ebase + g * _CH
        pltpu.make_async_copy(hw3_hbm.at[si[x]], a[x], smga[x]).wait()
        pltpu.make_async_copy(hw3_hbm.at[di[x]], b[x], smgb[x]).wait()

        def _arow(r2, cc):
            for rr in range(2):
                r = r2 * 2 + rr
                for j in range(_GP // _L):
                    sl = pl.ds(j * _L, _L)
                    gn[r, sl] = a[x][r, sl] + b[x][r, sl]
            return cc
        lax.fori_loop(0, _CH // 2, _arow, 0)
        pltpu.sync_copy(gn, out_hbm.at[pl.ds(pl.multiple_of(base, 8), _CH)])

    issue(0, 0)
    gather(0)
    issue(1, 1)
    gather(1)
    issue(2, 2)
    gather(2)

    def body(i, carry):
        g = 3 * i
        process(0, g)
        issue(0, g + 3)
        gather(0)
        process(1, g + 1)
        issue(1, g + 4)
        gather(1)
        process(2, g + 2)
        issue(2, g + 5)
        gather(2)
        return carry
    lax.fori_loop(0, (_NCHUNK - 5) // 3, body, 0)
    process(0, _NCHUNK - 5)
    issue(0, _NCHUNK - 2)
    gather(0)
    process(1, _NCHUNK - 4)
    issue(1, _NCHUNK - 1)
    gather(1)
    process(2, _NCHUNK - 3)
    process(0, _NCHUNK - 2)
    process(1, _NCHUNK - 1)


@functools.partial(
    pl.kernel,
    out_type=jax.ShapeDtypeStruct((_NC * _NP, _F), jnp.float32),
    mesh=_MESH,
    scratch_types=[
        pltpu.VMEM((_CH,), jnp.int32),
        pltpu.VMEM((_CH,), jnp.int32),
        pltpu.VMEM((_CH // 8, _F), jnp.float32),
        pltpu.VMEM((_CH // 8, _F), jnp.float32),
        pltpu.VMEM((_CH, _F), jnp.float32),
        pltpu.VMEM((_CH, _F), jnp.float32),
        pltpu.VMEM_SHARED((_NP, _F), jnp.float32),
    ] + [pltpu.SemaphoreType.DMA] * 6,
)
def _scatter_f(fe8_hbm, dst_hbm, out_hbm, di0, di1, fb0, fb1, u0, u1, acc,
               smd0, smd1, smf0, smf1, sms0, sms1):
    c = lax.axis_index("c")
    s = lax.axis_index("s")
    wid = s * _NC + c
    di = (di0, di1)
    fb = (fb0, fb1)
    u = (u0, u1)
    smd = (smd0, smd1)
    smf = (smf0, smf1)
    sms = (sms0, sms1)

    _zero_acc(u0, acc, s, _CH)

    def _zrow(r, carry):
        for j in range(_F // _L):
            u1[r, pl.ds(j * _L, _L)] = jnp.zeros((_L,), jnp.float32)
        return carry
    lax.fori_loop(0, _CH, _zrow, 0)
    plsc.subcore_barrier()

    ebase = wid * _EPW

    def drain_scatter(x):
        pltpu.make_async_copy(u[x], acc.at[di[x]], sms[x]).wait()

    def issue(x, g, drain):
        if drain:
            drain_scatter(x)
        base = ebase + g * _CH
        pltpu.async_copy(dst_hbm.at[pl.ds(base, _CH)], di[x], smd[x])
        pltpu.async_copy(
            fe8_hbm.at[pl.ds(pl.multiple_of(base // 8, 8), _CH // 8)], fb[x], smf[x])

    def process(x):
        pltpu.make_async_copy(fe8_hbm.at[pl.ds(0, _CH // 8)], fb[x], smf[x]).wait()

        def _expand(q, cc):
            for rr in range(8):
                u[x][q * 8 + rr, pl.ds(0, _FW)] = fb[x][q, pl.ds(rr * _FW, _FW)]
            return cc
        lax.fori_loop(0, _CH // 8, _expand, 0)
        pltpu.make_async_copy(dst_hbm.at[pl.ds(0, _CH)], di[x], smd[x]).wait()
        pltpu.async_copy(u[x], acc.at[di[x]], sms[x], add=True)

    issue(0, 0, False)
    issue(1, 1, False)

    def body(i, carry):
        g = 2 * i
        process(0)
        issue(0, g + 2, True)
        process(1)
        issue(1, g + 3, True)
        return carry
    lax.fori_loop(0, _NCHUNK // 2 - 1, body, 0)
    process(0)
    process(1)
    drain_scatter(0)
    drain_scatter(1)

    plsc.subcore_barrier()
    _acc_out(acc, out_hbm, c, s)


_NGRID = _NP // _BN
_EGRID = _EP // _BE


def _full(shape):
    return pl.BlockSpec(shape, lambda i: tuple(0 for _ in shape))


def _rows(shape):
    return pl.BlockSpec(shape, lambda i: (i,) + tuple(0 for _ in shape[1:]))


def _init_nodes(zp, embp):
    return pl.pallas_call(
        _init_nodes_body,
        grid=(_NGRID,),
        in_specs=[_rows((_BN, 1)), _full((128, _F))],
        out_specs=_rows((_BN, _F)),
        out_shape=jax.ShapeDtypeStruct((_NP, _F), jnp.float32),
    )(zp, embp)


def _init_edges(dp, offp, widp):
    return pl.pallas_call(
        _init_edges_body,
        grid=(_EGRID,),
        in_specs=[_rows((_BE, 1)), _full((1, _GP)), _full((1, _GP))],
        out_specs=_rows((_BE, _GP)),
        out_shape=jax.ShapeDtypeStruct((_EP, _GP), jnp.float32),
    )(dp, offp, widp)


def _node_mm(h, w1):
    return pl.pallas_call(
        _node_mm_body,
        grid=(_NGRID,),
        in_specs=[_rows((_BN, _F)), _full((_F, _F))],
        out_specs=_rows((_BN, _F)),
        out_shape=jax.ShapeDtypeStruct((_NP, _F), jnp.float32),
    )(h, w1)


def _edge_mm(e, w2, w4, condp):
    return pl.pallas_call(
        _edge_mm_body,
        grid=(_EGRID,),
        in_specs=[_rows((_BE, _GP)), _full((_GP, _F)), _full((_GP, _GP)),
                  _rows((_BE, 1))],
        out_specs=[_rows((_BE, _F)), _rows((_BE, _GP))],
        out_shape=[jax.ShapeDtypeStruct((_EP, _F), jnp.float32),
                   jax.ShapeDtypeStruct((_EP, _GP), jnp.float32)],
    )(e, w2, w4, condp)


def _node_upd(h, aggp, w3):
    a0 = pl.BlockSpec((_BN, _F), lambda i: (i, 0))
    a1 = pl.BlockSpec((_BN, _F), lambda i: (i + _NGRID, 0))
    return pl.pallas_call(
        _node_upd_body,
        grid=(_NGRID,),
        in_specs=[_rows((_BN, _F)), a0, a1, _full((_F, _F))],
        out_specs=[_rows((_BN, _F)), _rows((_BN, _F))],
        out_shape=[jax.ShapeDtypeStruct((_NP, _F), jnp.float32),
                   jax.ShapeDtypeStruct((_NP, _F), jnp.float32)],
    )(h, aggp, aggp, w3)


def _edge_upd(e, g, ew4):
    return pl.pallas_call(
        _edge_upd_body,
        grid=(_EGRID,),
        in_specs=[_rows((_BE, _GP))] * 3,
        out_specs=_rows((_BE, _GP)),
        out_shape=jax.ShapeDtypeStruct((_EP, _GP), jnp.float32),
    )(e, g, ew4)


def _force(e, wo1, bo1, wo2t, bo2, uvp):
    return pl.pallas_call(
        _force_body,
        grid=(_EGRID,),
        in_specs=[_rows((_BE, _GP)), _full((_GP, _GP)), _full((1, _GP)),
                  _full((1, _GP)), _full((1, 1)), _rows((_BE, _FW))],
        out_specs=_rows((_BE, _FW)),
        out_shape=jax.ShapeDtypeStruct((_EP, _FW), jnp.float32),
    )(e, wo1, bo1, wo2t, bo2, uvp)


def _combine(fp):
    p0 = pl.BlockSpec((_BN, _F), lambda i: (i, 0))
    p1 = pl.BlockSpec((_BN, _F), lambda i: (i + _NGRID, 0))
    return pl.pallas_call(
        _combine_body,
        grid=(_NGRID,),
        in_specs=[p0, p1],
        out_specs=_rows((_BN, _F)),
        out_shape=jax.ShapeDtypeStruct((_NP, _F), jnp.float32),
    )(fp, fp)


def kernel(Z, edge_index, distances, unit_vecs, conductance, emb_table,
           g_offsets, g_widths, W1, W2, W3, W4, Wo1, bo1, Wo2, bo2):
    f32 = jnp.float32
    epad = _EP - _E
    npad = _NP - _N

    src = jnp.pad(edge_index[0].astype(jnp.int32), (0, epad))
    dst = jnp.pad(edge_index[1].astype(jnp.int32), (0, epad))
    dp = jnp.pad(distances.astype(f32), (0, epad),
                 constant_values=1.0).reshape(_EP, 1)
    condp = jnp.pad(conductance.astype(f32), (0, epad)).reshape(_EP, 1)
    uvp = jnp.pad(unit_vecs.astype(f32), ((0, epad), (0, _FW - 3)))
    zp = jnp.pad(Z.astype(jnp.int32), (0, npad)).reshape(_NP, 1)
    embp = jnp.pad(emb_table.astype(f32), ((0, 128 - emb_table.shape[0]), (0, 0)))
    offp = jnp.pad(g_offsets.astype(f32), (0, _GP - _G)).reshape(1, _GP)
    widp = jnp.pad(g_widths.astype(f32), (0, _GP - _G),
                   constant_values=1.0).reshape(1, _GP)
    W1 = W1.astype(f32)
    W2p = jnp.pad(W2.astype(f32), ((0, 0), (0, _GP - _G), (0, 0)))
    W3p = jnp.pad(W3.astype(f32), ((0, 0), (0, 0), (0, _F - _G)))
    W4p = jnp.pad(W4.astype(f32), ((0, 0), (0, _GP - _G), (0, _GP - _G)))
    Wo1p = jnp.pad(Wo1.astype(f32), ((0, _GP - _G), (0, _GP - _G)))
    bo1p = jnp.pad(bo1.astype(f32), (0, _GP - _G)).reshape(1, _GP)
    Wo2t = jnp.pad(Wo2.astype(f32), ((0, _GP - _G), (0, 0))).reshape(1, _GP)
    bo2p = bo2.astype(f32).reshape(1, 1)

    h = _init_nodes(zp, embp)
    e = _init_edges(dp, offp, widp)

    for l in range(_NMP):
        sw1 = _node_mm(h, W1[l])
        t, ew4 = _edge_mm(e, W2p[l], W4p[l], condp)
        aggp = _msg_agg(sw1, t, src, dst)
        h, hw3 = _node_upd(h, aggp, W3p[l])
        g = _gather2(hw3, src, dst)
        e = _edge_upd(e, g, ew4)

    fe = _force(e, Wo1p, bo1p, Wo2t, bo2p, uvp)
    fp = _scatter_f(fe.reshape(_EP // 8, _F), dst)
    forces = _combine(fp)
    return forces[:_N, :3]


# speedup vs baseline: 1.6559x; 1.0001x over previous
import functools

import jax
import jax.numpy as jnp
from jax import lax
from jax.experimental import pallas as pl
from jax.experimental.pallas import tpu as pltpu
from jax.experimental.pallas import tpu_sc as plsc

_N = 10000
_E = 320000
_F = 128
_G = 20
_GP = 32
_NMP = 3
_NC = 2
_NS = 16
_L = 16
_NW = _NC * _NS
_NP = 10240
_EP = 327680
_CH = 128
_CHM = 80
_EPW = _EP // _NW
_NCHUNK = _EPW // _CH
_NCHUNKM = _EPW // _CHM
_RPS = _NP // _NS
_FW = 16
_LN2 = 0.6931471805599453
_BN = 512
_BE = 2048


def _ssp(x):
    return jnp.maximum(x, 0.0) + jnp.log1p(jnp.exp(-jnp.abs(x))) - _LN2


def _init_nodes_body(z_ref, emb_ref, h_ref):
    z = z_ref[...]
    oh = (z == lax.broadcasted_iota(jnp.int32, (_BN, 128), 1)).astype(jnp.float32)
    h_ref[...] = jnp.dot(oh, emb_ref[...], preferred_element_type=jnp.float32)


def _init_edges_body(d_ref, off_ref, w_ref, e_ref):
    d = d_ref[...]
    off = off_ref[...]
    w = w_ref[...]
    e = jnp.exp(-((d - off) ** 2) / (2.0 * w * w))
    mask = lax.broadcasted_iota(jnp.int32, e.shape, 1) < _G
    e_ref[...] = jnp.where(mask, e, 0.0)


def _node_mm_body(h_ref, w1_ref, sw1_ref):
    sw1_ref[...] = _ssp(
        jnp.dot(h_ref[...], w1_ref[...], preferred_element_type=jnp.float32))


def _edge_mm_body(e_ref, w2_ref, w4_ref, c_ref, t_ref, ew4_ref):
    e = e_ref[...]
    t_ref[...] = _ssp(
        jnp.dot(e, w2_ref[...], preferred_element_type=jnp.float32)) * c_ref[...]
    ew4_ref[...] = jnp.dot(e, w4_ref[...], preferred_element_type=jnp.float32)


def _node_upd_body(h_ref, a0_ref, a1_ref, w3_ref, hn_ref, hw3_ref):
    h = h_ref[...] + _ssp(a0_ref[...] + a1_ref[...])
    hn_ref[...] = h
    hw3_ref[...] = jnp.dot(h, w3_ref[...], preferred_element_type=jnp.float32)


def _edge_upd_body(e_ref, g_ref, ew4_ref, en_ref):
    en_ref[...] = e_ref[...] + _ssp(g_ref[...] + ew4_ref[...])


def _force_body(e_ref, wo1_ref, bo1_ref, wo2_ref, bo2_ref, uv_ref, fe_ref):
    z1 = _ssp(
        jnp.dot(e_ref[...], wo1_ref[...], preferred_element_type=jnp.float32)
        + bo1_ref[...])
    f = jnp.sum(z1 * wo2_ref[...], axis=1, keepdims=True) + bo2_ref[...]
    fe_ref[...] = f * uv_ref[...]


def _combine_body(p0_ref, p1_ref, o_ref):
    o_ref[...] = p0_ref[...] + p1_ref[...]


_MESH = plsc.VectorSubcoreMesh(
    core_axis_name="c", subcore_axis_name="s", num_cores=_NC, num_subcores=_NS)


def _zero_acc(u, acc, s, ch, sem):
    def _zrow(r, carry):
        for j in range(_F // _L):
            u[r, pl.ds(j * _L, _L)] = jnp.zeros((_L,), jnp.float32)
        return carry
    lax.fori_loop(0, ch, _zrow, 0)

    def _zacc(i, carry):
        pltpu.async_copy(
            u, acc.at[pl.ds(pl.multiple_of(s * _RPS + i * ch, 8), ch)], sem)
        return carry
    lax.fori_loop(0, _RPS // ch, _zacc, 0)

    def _zwait(i, carry):
        pltpu.make_async_copy(u, acc.at[pl.ds(0, ch)], sem).wait()
        return carry
    lax.fori_loop(0, _RPS // ch, _zwait, 0)


def _acc_out(acc, out_hbm, c, s, sem):
    def _wout(i, carry):
        r0 = s * _RPS + i * _CH
        pltpu.async_copy(acc.at[pl.ds(pl.multiple_of(r0, 8), _CH)],
                         out_hbm.at[pl.ds(pl.multiple_of(c * _NP + r0, 8), _CH)],
                         sem)
        return carry
    lax.fori_loop(0, _RPS // _CH, _wout, 0)

    def _wwait(i, carry):
        pltpu.make_async_copy(acc.at[pl.ds(0, _CH)],
                              out_hbm.at[pl.ds(0, _CH)], sem).wait()
        return carry
    lax.fori_loop(0, _RPS // _CH, _wwait, 0)


@functools.partial(
    pl.kernel,
    out_type=jax.ShapeDtypeStruct((_NC * _NP, _F), jnp.float32),
    mesh=_MESH,
    scratch_types=[
        pltpu.VMEM((_CHM,), jnp.int32),
        pltpu.VMEM((_CHM,), jnp.int32),
        pltpu.VMEM((_CHM,), jnp.int32),
        pltpu.VMEM((_CHM,), jnp.int32),
        pltpu.VMEM((_CHM, _F), jnp.float32),
        pltpu.VMEM((_CHM, _F), jnp.float32),
        pltpu.VMEM((_CHM, _F), jnp.float32),
        pltpu.VMEM((_CHM, _F), jnp.float32),
        pltpu.VMEM_SHARED((_NP, _F), jnp.float32),
    ] + [pltpu.SemaphoreType.DMA] * 10,
)
def _msg_agg(sw1_hbm, t_hbm, src_hbm, dst_hbm, out_hbm,
             si0, si1, di0, di1, u0, u1, t0, t1, acc,
             smi0, smi1, smd0, smd1, smt0, smt1, smg0, smg1, sms0, sms1):
    c = lax.axis_index("c")
    s = lax.axis_index("s")
    wid = s * _NC + c
    ebase = wid * _EPW
    si = (si0, si1)
    di = (di0, di1)
    u = (u0, u1)
    t = (t0, t1)
    smi = (smi0, smi1)
    smd = (smd0, smd1)
    smt = (smt0, smt1)
    smg = (smg0, smg1)
    sms = (sms0, sms1)

    _zero_acc(u0, acc, s, _CHM, smg0)
    plsc.subcore_barrier()

    def drain_scatter(x):
        pltpu.make_async_copy(u[x], acc.at[di[x]], sms[x]).wait()

    def issue(x, g, drain):
        if drain:
            drain_scatter(x)
        base = ebase + g * _CHM
        pltpu.async_copy(src_hbm.at[pl.ds(base, _CHM)], si[x], smi[x])
        pltpu.async_copy(dst_hbm.at[pl.ds(base, _CHM)], di[x], smd[x])
        pltpu.async_copy(t_hbm.at[pl.ds(pl.multiple_of(base, 8), _CHM)], t[x], smt[x])

    def gather(x):
        pltpu.make_async_copy(src_hbm.at[pl.ds(0, _CHM)], si[x], smi[x]).wait()
        pltpu.async_copy(sw1_hbm.at[si[x]], u[x], smg[x])

    def process(x):
        pltpu.make_async_copy(sw1_hbm.at[si[x]], u[x], smg[x]).wait()
        pltpu.make_async_copy(t_hbm.at[pl.ds(0, _CHM)], t[x], smt[x]).wait()

        def _mrow(r2, cc):
            for rr in range(2):
                r = r2 * 2 + rr
                for j in range(_F // _L):
                    sl = pl.ds(j * _L, _L)
                    u[x][r, sl] = u[x][r, sl] * t[x][r, sl]
            return cc
        lax.fori_loop(0, _CHM // 2, _mrow, 0)
        pltpu.make_async_copy(dst_hbm.at[pl.ds(0, _CHM)], di[x], smd[x]).wait()
        pltpu.async_copy(u[x], acc.at[di[x]], sms[x], add=True)

    issue(0, 0, False)
    gather(0)
    issue(1, 1, False)

    def body(i, carry):
        g = 2 * i
        process(0)
        gather(1)
        issue(0, g + 2, True)
        process(1)
        gather(0)
        issue(1, g + 3, True)
        return carry
    lax.fori_loop(0, _NCHUNKM // 2 - 1, body, 0)
    process(0)
    gather(1)
    process(1)
    drain_scatter(0)
    drain_scatter(1)

    plsc.subcore_barrier()
    _acc_out(acc, out_hbm, c, s, smg0)


@functools.partial(
    pl.kernel,
    out_type=jax.ShapeDtypeStruct((_EP, _GP), jnp.float32),
    mesh=_MESH,
    scratch_types=[
        pltpu.VMEM((_CH,), jnp.int32),
        pltpu.VMEM((_CH,), jnp.int32),
        pltpu.VMEM((_CH,), jnp.int32),
        pltpu.VMEM((_CH,), jnp.int32),
        pltpu.VMEM((_CH,), jnp.int32),
        pltpu.VMEM((_CH,), jnp.int32),
        pltpu.VMEM((_CH, _F), jnp.float32),
        pltpu.VMEM((_CH, _F), jnp.float32),
        pltpu.VMEM((_CH, _F), jnp.float32),
        pltpu.VMEM((_CH, _F), jnp.float32),
        pltpu.VMEM((_CH, _F), jnp.float32),
        pltpu.VMEM((_CH, _F), jnp.float32),
        pltpu.VMEM((_CH, _GP), jnp.float32),
    ] + [pltpu.SemaphoreType.DMA] * 12,
)
def _gather2(hw3_hbm, src_hbm, dst_hbm, out_hbm,
             si0, si1, si2, di0, di1, di2, a0, a1, a2, b0, b1, b2, gn,
             smi0, smi1, smi2, smd0, smd1, smd2,
             smga0, smga1, smga2, smgb0, smgb1, smgb2):
    c = lax.axis_index("c")
    s = lax.axis_index("s")
    wid = s * _NC + c
    ebase = wid * _EPW
    si = (si0, si1, si2)
    di = (di0, di1, di2)
    a = (a0, a1, a2)
    b = (b0, b1, b2)
    smi = (smi0, smi1, smi2)
    smd = (smd0, smd1, smd2)
    smga = (smga0, smga1, smga2)
    smgb = (smgb0, smgb1, smgb2)

    def issue(x, g):
        base = ebase + g * _CH
        pltpu.async_copy(src_hbm.at[pl.ds(base, _CH)], si[x], smi[x])
        pltpu.async_copy(dst_hbm.at[pl.ds(base, _CH)], di[x], smd[x])

    def gather(x):
        pltpu.make_async_copy(src_hbm.at[pl.ds(0, _CH)], si[x], smi[x]).wait()
        pltpu.async_copy(hw3_hbm.at[si[x]], a[x], smga[x])
        pltpu.make_async_copy(dst_hbm.at[pl.ds(0, _CH)], di[x], smd[x]).wait()
        pltpu.async_copy(hw3_hbm.at[di[x]], b[x], smgb[x])

    def process(x, g):
        base = ebase + g * _CH
        pltpu.make_async_copy(hw3_hbm.at[si[x]], a[x], smga[x]).wait()
        pltpu.make_async_copy(hw3_hbm.at[di[x]], b[x], smgb[x]).wait()

        def _arow(r2, cc):
            for rr in range(2):
                r = r2 * 2 + rr
                for j in range(_GP // _L):
                    sl = pl.ds(j * _L, _L)
                    gn[r, sl] = a[x][r, sl] + b[x][r, sl]
            return cc
        lax.fori_loop(0, _CH // 2, _arow, 0)
        pltpu.sync_copy(gn, out_hbm.at[pl.ds(pl.multiple_of(base, 8), _CH)])

    issue(0, 0)
    gather(0)
    issue(1, 1)
    gather(1)
    issue(2, 2)
    gather(2)

    def body(i, carry):
        g = 3 * i
        process(0, g)
        issue(0, g + 3)
        gather(0)
        process(1, g + 1)
        issue(1, g + 4)
        gather(1)
        process(2, g + 2)
        issue(2, g + 5)
        gather(2)
        return carry
    lax.fori_loop(0, (_NCHUNK - 5) // 3, body, 0)
    process(0, _NCHUNK - 5)
    issue(0, _NCHUNK - 2)
    gather(0)
    process(1, _NCHUNK - 4)
    issue(1, _NCHUNK - 1)
    gather(1)
    process(2, _NCHUNK - 3)
    process(0, _NCHUNK - 2)
    process(1, _NCHUNK - 1)


@functools.partial(
    pl.kernel,
    out_type=jax.ShapeDtypeStruct((_NC * _NP, _F), jnp.float32),
    mesh=_MESH,
    scratch_types=[
        pltpu.VMEM((_CH,), jnp.int32),
        pltpu.VMEM((_CH,), jnp.int32),
        pltpu.VMEM((_CH // 8, _F), jnp.float32),
        pltpu.VMEM((_CH // 8, _F), jnp.float32),
        pltpu.VMEM((_CH, _F), jnp.float32),
        pltpu.VMEM((_CH, _F), jnp.float32),
        pltpu.VMEM_SHARED((_NP, _F), jnp.float32),
    ] + [pltpu.SemaphoreType.DMA] * 6,
)
def _scatter_f(fe8_hbm, dst_hbm, out_hbm, di0, di1, fb0, fb1, u0, u1, acc,
               smd0, smd1, smf0, smf1, sms0, sms1):
    c = lax.axis_index("c")
    s = lax.axis_index("s")
    wid = s * _NC + c
    di = (di0, di1)
    fb = (fb0, fb1)
    u = (u0, u1)
    smd = (smd0, smd1)
    smf = (smf0, smf1)
    sms = (sms0, sms1)

    _zero_acc(u0, acc, s, _CH, smf0)

    def _zrow(r, carry):
        for j in range(_F // _L):
            u1[r, pl.ds(j * _L, _L)] = jnp.zeros((_L,), jnp.float32)
        return carry
    lax.fori_loop(0, _CH, _zrow, 0)
    plsc.subcore_barrier()

    ebase = wid * _EPW

    def drain_scatter(x):
        pltpu.make_async_copy(u[x], acc.at[di[x]], sms[x]).wait()

    def issue(x, g, drain):
        if drain:
            drain_scatter(x)
        base = ebase + g * _CH
        pltpu.async_copy(dst_hbm.at[pl.ds(base, _CH)], di[x], smd[x])
        pltpu.async_copy(
            fe8_hbm.at[pl.ds(pl.multiple_of(base // 8, 8), _CH // 8)], fb[x], smf[x])

    def process(x):
        pltpu.make_async_copy(fe8_hbm.at[pl.ds(0, _CH // 8)], fb[x], smf[x]).wait()

        def _expand(q, cc):
            for rr in range(8):
                u[x][q * 8 + rr, pl.ds(0, _FW)] = fb[x][q, pl.ds(rr * _FW, _FW)]
            return cc
        lax.fori_loop(0, _CH // 8, _expand, 0)
        pltpu.make_async_copy(dst_hbm.at[pl.ds(0, _CH)], di[x], smd[x]).wait()
        pltpu.async_copy(u[x], acc.at[di[x]], sms[x], add=True)

    issue(0, 0, False)
    issue(1, 1, False)

    def body(i, carry):
        g = 2 * i
        process(0)
        issue(0, g + 2, True)
        process(1)
        issue(1, g + 3, True)
        return carry
    lax.fori_loop(0, _NCHUNK // 2 - 1, body, 0)
    process(0)
    process(1)
    drain_scatter(0)
    drain_scatter(1)

    plsc.subcore_barrier()
    _acc_out(acc, out_hbm, c, s, smf0)


_NGRID = _NP // _BN
_EGRID = _EP // _BE


def _full(shape):
    return pl.BlockSpec(shape, lambda i: tuple(0 for _ in shape))


def _rows(shape):
    return pl.BlockSpec(shape, lambda i: (i,) + tuple(0 for _ in shape[1:]))


def _init_nodes(zp, embp):
    return pl.pallas_call(
        _init_nodes_body,
        grid=(_NGRID,),
        in_specs=[_rows((_BN, 1)), _full((128, _F))],
        out_specs=_rows((_BN, _F)),
        out_shape=jax.ShapeDtypeStruct((_NP, _F), jnp.float32),
    )(zp, embp)


def _init_edges(dp, offp, widp):
    return pl.pallas_call(
        _init_edges_body,
        grid=(_EGRID,),
        in_specs=[_rows((_BE, 1)), _full((1, _GP)), _full((1, _GP))],
        out_specs=_rows((_BE, _GP)),
        out_shape=jax.ShapeDtypeStruct((_EP, _GP), jnp.float32),
    )(dp, offp, widp)


def _node_mm(h, w1):
    return pl.pallas_call(
        _node_mm_body,
        grid=(_NGRID,),
        in_specs=[_rows((_BN, _F)), _full((_F, _F))],
        out_specs=_rows((_BN, _F)),
        out_shape=jax.ShapeDtypeStruct((_NP, _F), jnp.float32),
    )(h, w1)


def _edge_mm(e, w2, w4, condp):
    return pl.pallas_call(
        _edge_mm_body,
        grid=(_EGRID,),
        in_specs=[_rows((_BE, _GP)), _full((_GP, _F)), _full((_GP, _GP)),
                  _rows((_BE, 1))],
        out_specs=[_rows((_BE, _F)), _rows((_BE, _GP))],
        out_shape=[jax.ShapeDtypeStruct((_EP, _F), jnp.float32),
                   jax.ShapeDtypeStruct((_EP, _GP), jnp.float32)],
    )(e, w2, w4, condp)


def _node_upd(h, aggp, w3):
    a0 = pl.BlockSpec((_BN, _F), lambda i: (i, 0))
    a1 = pl.BlockSpec((_BN, _F), lambda i: (i + _NGRID, 0))
    return pl.pallas_call(
        _node_upd_body,
        grid=(_NGRID,),
        in_specs=[_rows((_BN, _F)), a0, a1, _full((_F, _F))],
        out_specs=[_rows((_BN, _F)), _rows((_BN, _F))],
        out_shape=[jax.ShapeDtypeStruct((_NP, _F), jnp.float32),
                   jax.ShapeDtypeStruct((_NP, _F), jnp.float32)],
    )(h, aggp, aggp, w3)


def _edge_upd(e, g, ew4):
    return pl.pallas_call(
        _edge_upd_body,
        grid=(_EGRID,),
        in_specs=[_rows((_BE, _GP))] * 3,
        out_specs=_rows((_BE, _GP)),
        out_shape=jax.ShapeDtypeStruct((_EP, _GP), jnp.float32),
    )(e, g, ew4)


def _force(e, wo1, bo1, wo2t, bo2, uvp):
    return pl.pallas_call(
        _force_body,
        grid=(_EGRID,),
        in_specs=[_rows((_BE, _GP)), _full((_GP, _GP)), _full((1, _GP)),
                  _full((1, _GP)), _full((1, 1)), _rows((_BE, _FW))],
        out_specs=_rows((_BE, _FW)),
        out_shape=jax.ShapeDtypeStruct((_EP, _FW), jnp.float32),
    )(e, wo1, bo1, wo2t, bo2, uvp)


def _combine(fp):
    p0 = pl.BlockSpec((_BN, _F), lambda i: (i, 0))
    p1 = pl.BlockSpec((_BN, _F), lambda i: (i + _NGRID, 0))
    return pl.pallas_call(
        _combine_body,
        grid=(_NGRID,),
        in_specs=[p0, p1],
        out_specs=_rows((_BN, _F)),
        out_shape=jax.ShapeDtypeStruct((_NP, _F), jnp.float32),
    )(fp, fp)


def kernel(Z, edge_index, distances, unit_vecs, conductance, emb_table,
           g_offsets, g_widths, W1, W2, W3, W4, Wo1, bo1, Wo2, bo2):
    f32 = jnp.float32
    epad = _EP - _E
    npad = _NP - _N

    src = jnp.pad(edge_index[0].astype(jnp.int32), (0, epad))
    dst = jnp.pad(edge_index[1].astype(jnp.int32), (0, epad))
    dp = jnp.pad(distances.astype(f32), (0, epad),
                 constant_values=1.0).reshape(_EP, 1)
    condp = jnp.pad(conductance.astype(f32), (0, epad)).reshape(_EP, 1)
    uvp = jnp.pad(unit_vecs.astype(f32), ((0, epad), (0, _FW - 3)))
    zp = jnp.pad(Z.astype(jnp.int32), (0, npad)).reshape(_NP, 1)
    embp = jnp.pad(emb_table.astype(f32), ((0, 128 - emb_table.shape[0]), (0, 0)))
    offp = jnp.pad(g_offsets.astype(f32), (0, _GP - _G)).reshape(1, _GP)
    widp = jnp.pad(g_widths.astype(f32), (0, _GP - _G),
                   constant_values=1.0).reshape(1, _GP)
    W1 = W1.astype(f32)
    W2p = jnp.pad(W2.astype(f32), ((0, 0), (0, _GP - _G), (0, 0)))
    W3p = jnp.pad(W3.astype(f32), ((0, 0), (0, 0), (0, _F - _G)))
    W4p = jnp.pad(W4.astype(f32), ((0, 0), (0, _GP - _G), (0, _GP - _G)))
    Wo1p = jnp.pad(Wo1.astype(f32), ((0, _GP - _G), (0, _GP - _G)))
    bo1p = jnp.pad(bo1.astype(f32), (0, _GP - _G)).reshape(1, _GP)
    Wo2t = jnp.pad(Wo2.astype(f32), ((0, _GP - _G), (0, 0))).reshape(1, _GP)
    bo2p = bo2.astype(f32).reshape(1, 1)

    h = _init_nodes(zp, embp)
    e = _init_edges(dp, offp, widp)

    for l in range(_NMP):
        sw1 = _node_mm(h, W1[l])
        t, ew4 = _edge_mm(e, W2p[l], W4p[l], condp)
        aggp = _msg_agg(sw1, t, src, dst)
        h, hw3 = _node_upd(h, aggp, W3p[l])
        g = _gather2(hw3, src, dst)
        e = _edge_upd(e, g, ew4)

    fe = _force(e, Wo1p, bo1p, Wo2t, bo2p, uvp)
    fp = _scatter_f(fe.reshape(_EP // 8, _F), dst)
    forces = _combine(fp)
    return forces[:_N, :3]
